# Initial kernel scaffold; baseline (speedup 1.0000x reference)
import functools
import math

import jax
import jax.numpy as jnp
from jax import lax
from jax.experimental import pallas as pl
from jax.experimental.pallas import tpu as pltpu
from jax.experimental.pallas import tpu_sc as plsc

N = 10000
E = 160000
D = 256
H = 8
DH = 32
HALF = D // 2

NC = 2
NS = 16
L = 16

ET = E // NS
EC = 80
NCHUNK = ET // EC
NPT = N // NS
NZ = 125

_F32 = jnp.float32
_I32 = jnp.int32


def _sc_edge_body(k_hbm, q_hbm, v_hbm, ee_hbm, src_hbm, dst_hbm,
                  eout_hbm, hatt_hbm, ssum_hbm,
                  src_v, dst_v, adjs_v, adjd_v, k_v, q_v, v_v, t_v, w_v,
                  zb_v, zbs_v, acc_sh, accs_sh, sem):
    s = lax.axis_index("s")
    c = lax.axis_index("c")

    @pl.loop(0, NZ)
    def _zero_rows(r):
        for j in range(HALF // L):
            zb_v[r, pl.ds(j * L, L)] = jnp.zeros((L,), _F32)
        zbs_v[r, :] = jnp.zeros((L,), _F32)

    row0 = s * NPT
    for b in range(NPT // NZ):
        pltpu.sync_copy(zb_v, acc_sh.at[pl.ds(row0 + b * NZ, NZ)])
        pltpu.sync_copy(zbs_v, accs_sh.at[pl.ds(row0 + b * NZ, NZ)])
    plsc.subcore_barrier()

    lanes = lax.iota(_I32, L)

    @pl.loop(0, NCHUNK)
    def _chunk(ch):
        base = s * ET + ch * EC
        pltpu.sync_copy(src_hbm.at[pl.ds(base, EC)], src_v)
        pltpu.sync_copy(dst_hbm.at[pl.ds(base, EC)], dst_v)

        off = c * N
        for g in range(EC // L):
            sl = pl.ds(g * L, L)
            adjs_v[sl] = src_v[sl] + off
            adjd_v[sl] = dst_v[sl] + off

        cp_k = pltpu.async_copy(k_hbm.at[adjs_v], k_v, sem)
        cp_q = pltpu.async_copy(q_hbm.at[adjd_v], q_v, sem)
        cp_v = pltpu.async_copy(v_hbm.at[adjd_v], v_v, sem)
        cp_e = pltpu.async_copy(ee_hbm.at[pl.ds(c * E + base, EC)], t_v, sem)
        cp_k.wait()
        cp_q.wait()
        cp_v.wait()
        cp_e.wait()

        @pl.loop(0, EC)
        def _tloop(ei):
            for j in range(HALF // L):
                sl = pl.ds(j * L, L)
                t_v[ei, sl] = k_v[ei, sl] * q_v[ei, sl] * t_v[ei, sl]

        cp_out = pltpu.async_copy(t_v, eout_hbm.at[pl.ds(c * E + base, EC)],
                                  sem)

        @pl.loop(0, EC // L)
        def _score(g):
            rows = g * L + lanes
            for hh in range(H // 2):
                acc = jnp.zeros((L,), _F32)
                for j in range(DH):
                    cols = jnp.full((L,), hh * DH + j, _I32)
                    acc = acc + plsc.load_gather(t_v, [rows, cols])
                w = jnp.exp(acc)
                plsc.store_scatter(w_v, [rows, jnp.full((L,), hh, _I32)], w)

        @pl.loop(0, EC)
        def _wv(ei):
            for hh in range(H // 2):
                bc = jnp.full((L,), w_v[ei, hh], _F32)
                for j2 in range(DH // L):
                    sl = pl.ds(hh * DH + j2 * L, L)
                    v_v[ei, sl] = v_v[ei, sl] * bc

        pltpu.sync_copy(v_v, acc_sh.at[src_v], add=True)
        pltpu.sync_copy(w_v, accs_sh.at[src_v], add=True)
        cp_out.wait()

    plsc.subcore_barrier()

    pltpu.sync_copy(acc_sh.at[pl.ds(row0, NPT)],
                    hatt_hbm.at[pl.ds(c * N + row0, NPT)])
    pltpu.sync_copy(accs_sh.at[pl.ds(row0, NPT)],
                    ssum_hbm.at[pl.ds(c * N + row0, NPT)])


def _sc_edge(k2, q2, v2, ee2, src, dst):
    mesh = plsc.VectorSubcoreMesh(core_axis_name="c", subcore_axis_name="s")
    fn = pl.kernel(
        _sc_edge_body,
        out_type=[
            jax.ShapeDtypeStruct((NC * E, HALF), _F32),
            jax.ShapeDtypeStruct((NC * N, HALF), _F32),
            jax.ShapeDtypeStruct((NC * N, L), _F32),
        ],
        mesh=mesh,
        scratch_types=[
            pltpu.VMEM((EC,), _I32),
            pltpu.VMEM((EC,), _I32),
            pltpu.VMEM((EC,), _I32),
            pltpu.VMEM((EC,), _I32),
            pltpu.VMEM((EC, HALF), _F32),
            pltpu.VMEM((EC, HALF), _F32),
            pltpu.VMEM((EC, HALF), _F32),
            pltpu.VMEM((EC, HALF), _F32),
            pltpu.VMEM((EC, L), _F32),
            pltpu.VMEM((NZ, HALF), _F32),
            pltpu.VMEM((NZ, L), _F32),
            pltpu.VMEM_SHARED((N, HALF), _F32),
            pltpu.VMEM_SHARED((N, L), _F32),
            pltpu.SemaphoreType.DMA,
        ],
    )
    return fn(k2, q2, v2, ee2, src, dst)


def _proj_body(x_ref, w_ref, o_ref, nout):
    y = jnp.dot(x_ref[...], w_ref[...], preferred_element_type=_F32)
    for k in range(nout):
        o_ref[k, :, :] = y[:, k * HALF:(k + 1) * HALF]


def _proj(x, w, bn):
    rows, _ = x.shape
    nout = w.shape[1] // HALF
    grid = rows // bn
    return pl.pallas_call(
        functools.partial(_proj_body, nout=nout),
        grid=(grid,),
        in_specs=[
            pl.BlockSpec((bn, D), lambda i: (i, 0)),
            pl.BlockSpec((D, nout * HALF), lambda i: (0, 0)),
        ],
        out_specs=pl.BlockSpec((nout, bn, HALF), lambda i: (0, i, 0)),
        out_shape=jax.ShapeDtypeStruct((nout, rows, HALF), _F32),
    )(x, w)


def _stats_update(ref_sum, ref_sq, y, first):
    @pl.when(first)
    def _init():
        ref_sum[...] = jnp.zeros_like(ref_sum)
        ref_sq[...] = jnp.zeros_like(ref_sq)

    cs = jnp.sum(y, axis=0, keepdims=True)
    cq = jnp.sum(y * y, axis=0, keepdims=True)
    ref_sum[...] = ref_sum[...] + jnp.broadcast_to(cs, ref_sum.shape)
    ref_sq[...] = ref_sq[...] + jnp.broadcast_to(cq, ref_sq.shape)


def _post_a_h_body(a0, a1, s0, s1, xin, w_ref, b_ref,
                   y_ref, sum_ref, sq_ref):
    col = lax.broadcasted_iota(_I32, (L, HALF), 1) // DH
    row = lax.broadcasted_iota(_I32, (L, HALF), 0)
    bmat = (col == row).astype(_F32)
    s0v = s0[...]
    s1v = s1[...]
    r0 = jnp.where(s0v > 0, 1.0 / jnp.where(s0v > 0, s0v, 1.0), 0.0)
    r1 = jnp.where(s1v > 0, 1.0 / jnp.where(s1v > 0, s1v, 1.0), 0.0)
    att0 = a0[...] * jnp.dot(r0, bmat, preferred_element_type=_F32)
    att1 = a1[...] * jnp.dot(r1, bmat, preferred_element_type=_F32)
    att = jnp.concatenate([att0, att1], axis=1)
    y = jnp.dot(att, w_ref[...], preferred_element_type=_F32)
    y = y + b_ref[...] + xin[...]
    y_ref[...] = y
    _stats_update(sum_ref, sq_ref, y, pl.program_id(0) == 0)


def _post_a_e_body(a0, a1, xin, w_ref, b_ref, y_ref, sum_ref, sq_ref):
    att = jnp.concatenate([a0[...], a1[...]], axis=1)
    y = jnp.dot(att, w_ref[...], preferred_element_type=_F32)
    y = y + b_ref[...] + xin[...]
    y_ref[...] = y
    _stats_update(sum_ref, sq_ref, y, pl.program_id(0) == 0)


def _bn(y, sum_ref, sq_ref, g_ref, b_ref, rows):
    mean = sum_ref[0:1, :] * (1.0 / rows)
    var = sq_ref[0:1, :] * (1.0 / rows) - mean * mean
    inv = lax.rsqrt(var + 1e-5)
    return (y - mean) * inv * g_ref[...] + b_ref[...]


def _post_b_body(y_in, sum_ref, sq_ref, g_ref, b_ref, w1_ref, b1_ref,
                 w2_ref, b2_ref, f_ref, sum2_ref, sq2_ref, rows):
    z = _bn(y_in[...], sum_ref, sq_ref, g_ref, b_ref, rows)
    u = jnp.dot(z, w1_ref[...], preferred_element_type=_F32) + b1_ref[...]
    u = jnp.maximum(u, 0.0)
    f = jnp.dot(u, w2_ref[...], preferred_element_type=_F32) + b2_ref[...] + z
    f_ref[...] = f
    _stats_update(sum2_ref, sq2_ref, f, pl.program_id(0) == 0)


def _post_c_body(f_in, sum_ref, sq_ref, g_ref, b_ref, o_ref, rows):
    o_ref[...] = _bn(f_in[...], sum_ref, sq_ref, g_ref, b_ref, rows)


def _full_spec(shape):
    return pl.BlockSpec(shape, lambda i: tuple(0 for _ in shape))


def _post_a_h(hatt2, ssum2, h_in, wo, bo, bn):
    grid = N // bn
    blk = lambda r0: pl.BlockSpec((bn, HALF), lambda i, r0=r0: (r0 + i, 0))
    blks = lambda r0: pl.BlockSpec((bn, L), lambda i, r0=r0: (r0 + i, 0))
    return pl.pallas_call(
        _post_a_h_body,
        grid=(grid,),
        in_specs=[
            blk(0), blk(N // bn), blks(0), blks(N // bn),
            pl.BlockSpec((bn, D), lambda i: (i, 0)),
            _full_spec((D, D)), _full_spec((1, D)),
        ],
        out_specs=[
            pl.BlockSpec((bn, D), lambda i: (i, 0)),
            _full_spec((8, D)), _full_spec((8, D)),
        ],
        out_shape=[
            jax.ShapeDtypeStruct((N, D), _F32),
            jax.ShapeDtypeStruct((8, D), _F32),
            jax.ShapeDtypeStruct((8, D), _F32),
        ],
    )(hatt2, hatt2, ssum2, ssum2, h_in, wo, bo)


def _post_a_e(eout2, e_in, wo, bo, bn):
    grid = E // bn
    blk = lambda r0: pl.BlockSpec((bn, HALF), lambda i, r0=r0: (r0 + i, 0))
    return pl.pallas_call(
        _post_a_e_body,
        grid=(grid,),
        in_specs=[
            blk(0), blk(E // bn),
            pl.BlockSpec((bn, D), lambda i: (i, 0)),
            _full_spec((D, D)), _full_spec((1, D)),
        ],
        out_specs=[
            pl.BlockSpec((bn, D), lambda i: (i, 0)),
            _full_spec((8, D)), _full_spec((8, D)),
        ],
        out_shape=[
            jax.ShapeDtypeStruct((E, D), _F32),
            jax.ShapeDtypeStruct((8, D), _F32),
            jax.ShapeDtypeStruct((8, D), _F32),
        ],
    )(eout2, eout2, e_in, wo, bo)


def _post_b(y, sums, sqs, g, b, w1, b1, w2, b2, bn):
    rows = y.shape[0]
    grid = rows // bn
    return pl.pallas_call(
        functools.partial(_post_b_body, rows=float(rows)),
        grid=(grid,),
        in_specs=[
            pl.BlockSpec((bn, D), lambda i: (i, 0)),
            _full_spec((8, D)), _full_spec((8, D)),
            _full_spec((1, D)), _full_spec((1, D)),
            _full_spec((D, 2 * D)), _full_spec((1, 2 * D)),
            _full_spec((2 * D, D)), _full_spec((1, D)),
        ],
        out_specs=[
            pl.BlockSpec((bn, D), lambda i: (i, 0)),
            _full_spec((8, D)), _full_spec((8, D)),
        ],
        out_shape=[
            jax.ShapeDtypeStruct((rows, D), _F32),
            jax.ShapeDtypeStruct((8, D), _F32),
            jax.ShapeDtypeStruct((8, D), _F32),
        ],
    )(y, sums, sqs, g, b, w1, b1, w2, b2)


def _post_c(f, sums, sqs, g, b, bn):
    rows = f.shape[0]
    grid = rows // bn
    return pl.pallas_call(
        functools.partial(_post_c_body, rows=float(rows)),
        grid=(grid,),
        in_specs=[
            pl.BlockSpec((bn, D), lambda i: (i, 0)),
            _full_spec((8, D)), _full_spec((8, D)),
            _full_spec((1, D)), _full_spec((1, D)),
        ],
        out_specs=pl.BlockSpec((bn, D), lambda i: (i, 0)),
        out_shape=jax.ShapeDtypeStruct((rows, D), _F32),
    )(f, sums, sqs, g, b)


def kernel(h, e, edge_index, WQ, WK, WV, WE, WOh, bOh, WOe, bOe,
           bn1h_g, bn1h_b, bn1e_g, bn1e_b, W1h, b1h, W2h, b2h,
           W1e, b1e, W2e, b2e, bn2h_g, bn2h_b, bn2e_g, bn2e_b):
    src = edge_index[0].astype(_I32)
    dst = edge_index[1].astype(_I32)
    scale = 1.0 / math.sqrt(DH)

    wq = WQ.T
    wk = WK.T * scale
    wv = WV.T
    wkqv = jnp.concatenate([wk, wq, wv], axis=1)

    kqv = _proj(h, wkqv, 1000)
    k2 = kqv[0:2].reshape(NC * N, HALF)
    q2 = kqv[2:4].reshape(NC * N, HALF)
    v2 = kqv[4:6].reshape(NC * N, HALF)
    ee2 = _proj(e, WE.T, 2000).reshape(NC * E, HALF)

    eout2, hatt2, ssum2 = _sc_edge(k2, q2, v2, ee2, src, dst)

    y_h, s1h, q1h = _post_a_h(hatt2, ssum2, h, WOh.T,
                              bOh.reshape(1, D), 1000)
    f_h, s2h, q2h = _post_b(y_h, s1h, q1h, bn1h_g.reshape(1, D),
                            bn1h_b.reshape(1, D), W1h.T, b1h.reshape(1, 2 * D),
                            W2h.T, b2h.reshape(1, D), 1000)
    h_out = _post_c(f_h, s2h, q2h, bn2h_g.reshape(1, D),
                    bn2h_b.reshape(1, D), 1000)

    y_e, s1e, q1e = _post_a_e(eout2, e, WOe.T, bOe.reshape(1, D), 2000)
    f_e, s2e, q2e = _post_b(y_e, s1e, q1e, bn1e_g.reshape(1, D),
                            bn1e_b.reshape(1, D), W1e.T, b1e.reshape(1, 2 * D),
                            W2e.T, b2e.reshape(1, D), 2000)
    e_out = _post_c(f_e, s2e, q2e, bn2e_g.reshape(1, D),
                    bn2e_b.reshape(1, D), 2000)

    return (h_out, e_out)


# speedup vs baseline: 3.3915x; 3.3915x over previous
import functools
import math

import jax
import jax.numpy as jnp
from jax import lax
from jax.experimental import pallas as pl
from jax.experimental.pallas import tpu as pltpu
from jax.experimental.pallas import tpu_sc as plsc

N = 10000
E = 160000
D = 256
H = 8
DH = 32
HALF = D // 2

NC = 2
NS = 16
L = 16

ET = E // NS
EC = 80
NCHUNK = ET // EC
RPT = 624
NZ = 208
NREM = N - NS * RPT

_F32 = jnp.float32
_I32 = jnp.int32


def _sc_edge_body(k_hbm, q_hbm, v_hbm, ee_hbm, src_hbm, dst_hbm,
                  z128_hbm,
                  eout_hbm, hatt_hbm, w_hbm,
                  src_v, dst_v, adjs_v, kq_v, t_v, w_v,
                  acc_sh, sem, sem_o):
    s = lax.axis_index("s")
    c = lax.axis_index("c")

    row0 = s * RPT
    pltpu.sync_copy(z128_hbm.at[pl.ds(row0, RPT)], acc_sh.at[pl.ds(row0, RPT)])

    @pl.when(s < 2)
    def _zero_rem():
        r = NS * RPT + s * 8
        pltpu.sync_copy(z128_hbm.at[pl.ds(r, 8)], acc_sh.at[pl.ds(r, 8)])

    plsc.subcore_barrier()

    lanes = lax.iota(_I32, L)

    @pl.loop(0, NCHUNK)
    def _chunk(ch):
        base = s * ET + ch * EC
        pltpu.sync_copy(src_hbm.at[pl.ds(base, EC)], src_v)
        pltpu.sync_copy(dst_hbm.at[pl.ds(base, EC)], dst_v)

        off = c * N
        for g in range(EC // L):
            sl = pl.ds(g * L, L)
            adjs_v[sl] = src_v[sl] + off
            dst_v[sl] = dst_v[sl] + off

        cp_k = pltpu.async_copy(k_hbm.at[adjs_v], kq_v, sem)
        cp_e = pltpu.async_copy(
            ee_hbm.at[pl.ds((c * E + base) * HALF, EC * HALF)], t_v, sem)
        cp_k.wait()
        cp_e.wait()

        @pl.loop(0, EC)
        def _tmul_k(ei):
            tbase = ei * HALF
            for j in range(HALF // L):
                tsl = pl.ds(tbase + j * L, L)
                t_v[tsl] = kq_v[ei, pl.ds(j * L, L)] * t_v[tsl]

        pltpu.async_copy(q_hbm.at[dst_v], kq_v, sem).wait()

        @pl.loop(0, EC)
        def _tmul_q(ei):
            tbase = ei * HALF
            for j in range(HALF // L):
                tsl = pl.ds(tbase + j * L, L)
                t_v[tsl] = kq_v[ei, pl.ds(j * L, L)] * t_v[tsl]

        cp_v = pltpu.async_copy(v_hbm.at[dst_v], kq_v, sem)
        cp_out = pltpu.async_copy(
            t_v, eout_hbm.at[pl.ds((c * E + base) * HALF, EC * HALF)], sem_o)

        perms = [(lanes ^ k) for k in (1, 2, 4, 8)]

        @pl.loop(0, EC)
        def _score(ei):
            tbase = ei * HALF
            wrow = jnp.zeros((L,), _F32)
            for hh in range(H // 2):
                a = t_v[pl.ds(tbase + 2 * hh * L, L)] + \
                    t_v[pl.ds(tbase + (2 * hh + 1) * L, L)]
                for p in perms:
                    a = a + a.at[p].get(mode="promise_in_bounds")
                wrow = jnp.where(lanes == hh, jnp.exp(a), wrow)
            w_v[ei, :] = wrow

        cp_v.wait()

        @pl.loop(0, EC)
        def _wv(ei):
            wrow = w_v[ei, :]
            for hh in range(H // 2):
                bc = jnp.full((L,), wrow[hh], _F32)
                for j2 in range(DH // L):
                    sl = pl.ds(hh * DH + j2 * L, L)
                    kq_v[ei, sl] = kq_v[ei, sl] * bc

        cp_w = pltpu.async_copy(w_v, w_hbm.at[pl.ds(c * E + base, EC)], sem_o)

        pltpu.sync_copy(kq_v, acc_sh.at[src_v], add=True)
        cp_out.wait()
        cp_w.wait()

    plsc.subcore_barrier()

    pltpu.sync_copy(acc_sh.at[pl.ds(row0, RPT)],
                    hatt_hbm.at[pl.ds(c * N + row0, RPT)])

    @pl.when(s < 2)
    def _dump_rem():
        r = NS * RPT + s * 8
        pltpu.sync_copy(acc_sh.at[pl.ds(r, 8)],
                        hatt_hbm.at[pl.ds(c * N + r, 8)])


def _sc_edge(k2, q2, v2, ee2, src, dst):
    mesh = plsc.VectorSubcoreMesh(core_axis_name="c", subcore_axis_name="s")
    fn = pl.kernel(
        _sc_edge_body,
        out_type=[
            jax.ShapeDtypeStruct((NC * E * HALF,), _F32),
            jax.ShapeDtypeStruct((NC * N, HALF), _F32),
            jax.ShapeDtypeStruct((NC * E, L), _F32),
        ],
        mesh=mesh,
        scratch_types=[
            pltpu.VMEM((EC,), _I32),
            pltpu.VMEM((EC,), _I32),
            pltpu.VMEM((EC,), _I32),
            pltpu.VMEM((EC, HALF), _F32),
            pltpu.VMEM((EC * HALF,), _F32),
            pltpu.VMEM((EC, L), _F32),
            pltpu.VMEM_SHARED((N, HALF), _F32),
            pltpu.SemaphoreType.DMA,
            pltpu.SemaphoreType.DMA,
        ],
    )
    z128 = jnp.zeros((N, HALF), _F32)
    return fn(k2, q2, v2, ee2, src, dst, z128)


def _sc_ssum_body(w_hbm, src_hbm, z128_hbm, ssum_hbm,
                  src_v, w_v, x_v, acc_sh, sem):
    s = lax.axis_index("s")
    c = lax.axis_index("c")

    row0 = s * RPT
    pltpu.sync_copy(z128_hbm.at[pl.ds(row0, RPT)], acc_sh.at[pl.ds(row0, RPT)])

    @pl.when(s < 2)
    def _zero_rem():
        r = NS * RPT + s * 8
        pltpu.sync_copy(z128_hbm.at[pl.ds(r, 8)], acc_sh.at[pl.ds(r, 8)])

    @pl.loop(0, EC)
    def _zero_x(ei):
        for j in range(HALF // L):
            x_v[ei, pl.ds(j * L, L)] = jnp.zeros((L,), _F32)

    plsc.subcore_barrier()

    @pl.loop(0, NCHUNK)
    def _chunk(ch):
        base = s * ET + ch * EC
        pltpu.sync_copy(src_hbm.at[pl.ds(base, EC)], src_v)
        pltpu.sync_copy(w_hbm.at[pl.ds(c * E + base, EC)], w_v)

        @pl.loop(0, EC)
        def _expand(ei):
            x_v[ei, pl.ds(0, L)] = w_v[ei, :]

        pltpu.sync_copy(x_v, acc_sh.at[src_v], add=True)

    plsc.subcore_barrier()

    pltpu.sync_copy(acc_sh.at[pl.ds(row0, RPT)],
                    ssum_hbm.at[pl.ds(c * N + row0, RPT)])

    @pl.when(s < 2)
    def _dump_rem():
        r = NS * RPT + s * 8
        pltpu.sync_copy(acc_sh.at[pl.ds(r, 8)],
                        ssum_hbm.at[pl.ds(c * N + r, 8)])


def _sc_ssum(w2, src):
    mesh = plsc.VectorSubcoreMesh(core_axis_name="c", subcore_axis_name="s")
    fn = pl.kernel(
        _sc_ssum_body,
        out_type=[jax.ShapeDtypeStruct((NC * N, HALF), _F32)],
        mesh=mesh,
        scratch_types=[
            pltpu.VMEM((EC,), _I32),
            pltpu.VMEM((EC, L), _F32),
            pltpu.VMEM((EC, HALF), _F32),
            pltpu.VMEM_SHARED((N, HALF), _F32),
            pltpu.SemaphoreType.DMA,
        ],
    )
    z128 = jnp.zeros((N, HALF), _F32)
    return fn(w2, src, z128)[0]


def _proj_body(x_ref, w_ref, o_ref, nout):
    y = jnp.dot(x_ref[...], w_ref[...], preferred_element_type=_F32)
    for k in range(nout):
        o_ref[k, :, :] = y[:, k * HALF:(k + 1) * HALF]


def _proj(x, w, bn):
    rows, _ = x.shape
    nout = w.shape[1] // HALF
    grid = rows // bn
    return pl.pallas_call(
        functools.partial(_proj_body, nout=nout),
        grid=(grid,),
        in_specs=[
            pl.BlockSpec((bn, D), lambda i: (i, 0)),
            pl.BlockSpec((D, nout * HALF), lambda i: (0, 0)),
        ],
        out_specs=pl.BlockSpec((nout, bn, HALF), lambda i: (0, i, 0)),
        out_shape=jax.ShapeDtypeStruct((nout, rows, HALF), _F32),
    )(x, w)


def _stats_update(ref_sum, ref_sq, y, first):
    @pl.when(first)
    def _init():
        ref_sum[...] = jnp.zeros_like(ref_sum)
        ref_sq[...] = jnp.zeros_like(ref_sq)

    cs = jnp.sum(y, axis=0, keepdims=True)
    cq = jnp.sum(y * y, axis=0, keepdims=True)
    ref_sum[...] = ref_sum[...] + jnp.broadcast_to(cs, ref_sum.shape)
    ref_sq[...] = ref_sq[...] + jnp.broadcast_to(cq, ref_sq.shape)


def _post_a_h_body(a0, a1, s0, s1, xin, w_ref, b_ref,
                   y_ref, sum_ref, sq_ref):
    col = lax.broadcasted_iota(_I32, (L, HALF), 1) // DH
    row = lax.broadcasted_iota(_I32, (L, HALF), 0)
    bmat = (col == row).astype(_F32)
    s0v = s0[:, :L]
    s1v = s1[:, :L]
    r0 = jnp.where(s0v > 0, 1.0 / jnp.where(s0v > 0, s0v, 1.0), 0.0)
    r1 = jnp.where(s1v > 0, 1.0 / jnp.where(s1v > 0, s1v, 1.0), 0.0)
    att0 = a0[...] * jnp.dot(r0, bmat, preferred_element_type=_F32)
    att1 = a1[...] * jnp.dot(r1, bmat, preferred_element_type=_F32)
    att = jnp.concatenate([att0, att1], axis=1)
    y = jnp.dot(att, w_ref[...], preferred_element_type=_F32)
    y = y + b_ref[...] + xin[...]
    y_ref[...] = y
    _stats_update(sum_ref, sq_ref, y, pl.program_id(0) == 0)


def _post_a_e_body(a0, a1, xin, w_ref, b_ref, y_ref, sum_ref, sq_ref):
    att = jnp.concatenate([a0[...], a1[...]], axis=1)
    y = jnp.dot(att, w_ref[...], preferred_element_type=_F32)
    y = y + b_ref[...] + xin[...]
    y_ref[...] = y
    _stats_update(sum_ref, sq_ref, y, pl.program_id(0) == 0)


def _bn(y, sum_ref, sq_ref, g_ref, b_ref, rows):
    mean = sum_ref[0:1, :] * (1.0 / rows)
    var = sq_ref[0:1, :] * (1.0 / rows) - mean * mean
    inv = lax.rsqrt(var + 1e-5)
    return (y - mean) * inv * g_ref[...] + b_ref[...]


def _post_b_body(y_in, sum_ref, sq_ref, g_ref, b_ref, w1_ref, b1_ref,
                 w2_ref, b2_ref, f_ref, sum2_ref, sq2_ref, rows):
    z = _bn(y_in[...], sum_ref, sq_ref, g_ref, b_ref, rows)
    u = jnp.dot(z, w1_ref[...], preferred_element_type=_F32) + b1_ref[...]
    u = jnp.maximum(u, 0.0)
    f = jnp.dot(u, w2_ref[...], preferred_element_type=_F32) + b2_ref[...] + z
    f_ref[...] = f
    _stats_update(sum2_ref, sq2_ref, f, pl.program_id(0) == 0)


def _post_c_body(f_in, sum_ref, sq_ref, g_ref, b_ref, o_ref, rows):
    o_ref[...] = _bn(f_in[...], sum_ref, sq_ref, g_ref, b_ref, rows)


def _full_spec(shape):
    return pl.BlockSpec(shape, lambda i: tuple(0 for _ in shape))


def _post_a_h(hatt2, ssum2, h_in, wo, bo, bn):
    grid = N // bn
    blk = lambda r0: pl.BlockSpec((bn, HALF), lambda i, r0=r0: (r0 + i, 0))
    blks = blk
    return pl.pallas_call(
        _post_a_h_body,
        grid=(grid,),
        in_specs=[
            blk(0), blk(N // bn), blks(0), blks(N // bn),
            pl.BlockSpec((bn, D), lambda i: (i, 0)),
            _full_spec((D, D)), _full_spec((1, D)),
        ],
        out_specs=[
            pl.BlockSpec((bn, D), lambda i: (i, 0)),
            _full_spec((8, D)), _full_spec((8, D)),
        ],
        out_shape=[
            jax.ShapeDtypeStruct((N, D), _F32),
            jax.ShapeDtypeStruct((8, D), _F32),
            jax.ShapeDtypeStruct((8, D), _F32),
        ],
    )(hatt2, hatt2, ssum2, ssum2, h_in, wo, bo)


def _post_a_e(eout2, e_in, wo, bo, bn):
    grid = E // bn
    blk = lambda r0: pl.BlockSpec((bn, HALF), lambda i, r0=r0: (r0 + i, 0))
    return pl.pallas_call(
        _post_a_e_body,
        grid=(grid,),
        in_specs=[
            blk(0), blk(E // bn),
            pl.BlockSpec((bn, D), lambda i: (i, 0)),
            _full_spec((D, D)), _full_spec((1, D)),
        ],
        out_specs=[
            pl.BlockSpec((bn, D), lambda i: (i, 0)),
            _full_spec((8, D)), _full_spec((8, D)),
        ],
        out_shape=[
            jax.ShapeDtypeStruct((E, D), _F32),
            jax.ShapeDtypeStruct((8, D), _F32),
            jax.ShapeDtypeStruct((8, D), _F32),
        ],
    )(eout2, eout2, e_in, wo, bo)


def _post_b(y, sums, sqs, g, b, w1, b1, w2, b2, bn):
    rows = y.shape[0]
    grid = rows // bn
    return pl.pallas_call(
        functools.partial(_post_b_body, rows=float(rows)),
        grid=(grid,),
        in_specs=[
            pl.BlockSpec((bn, D), lambda i: (i, 0)),
            _full_spec((8, D)), _full_spec((8, D)),
            _full_spec((1, D)), _full_spec((1, D)),
            _full_spec((D, 2 * D)), _full_spec((1, 2 * D)),
            _full_spec((2 * D, D)), _full_spec((1, D)),
        ],
        out_specs=[
            pl.BlockSpec((bn, D), lambda i: (i, 0)),
            _full_spec((8, D)), _full_spec((8, D)),
        ],
        out_shape=[
            jax.ShapeDtypeStruct((rows, D), _F32),
            jax.ShapeDtypeStruct((8, D), _F32),
            jax.ShapeDtypeStruct((8, D), _F32),
        ],
    )(y, sums, sqs, g, b, w1, b1, w2, b2)


def _post_c(f, sums, sqs, g, b, bn):
    rows = f.shape[0]
    grid = rows // bn
    return pl.pallas_call(
        functools.partial(_post_c_body, rows=float(rows)),
        grid=(grid,),
        in_specs=[
            pl.BlockSpec((bn, D), lambda i: (i, 0)),
            _full_spec((8, D)), _full_spec((8, D)),
            _full_spec((1, D)), _full_spec((1, D)),
        ],
        out_specs=pl.BlockSpec((bn, D), lambda i: (i, 0)),
        out_shape=jax.ShapeDtypeStruct((rows, D), _F32),
    )(f, sums, sqs, g, b)


def kernel(h, e, edge_index, WQ, WK, WV, WE, WOh, bOh, WOe, bOe,
           bn1h_g, bn1h_b, bn1e_g, bn1e_b, W1h, b1h, W2h, b2h,
           W1e, b1e, W2e, b2e, bn2h_g, bn2h_b, bn2e_g, bn2e_b):
    src = edge_index[0].astype(_I32)
    dst = edge_index[1].astype(_I32)
    scale = 1.0 / math.sqrt(DH)

    wq = WQ.T
    wk = WK.T * scale
    wv = WV.T
    wkqv = jnp.concatenate([wk, wq, wv], axis=1)

    kqv = _proj(h, wkqv, 1000)
    k2 = kqv[0:2].reshape(NC * N, HALF)
    q2 = kqv[2:4].reshape(NC * N, HALF)
    v2 = kqv[4:6].reshape(NC * N, HALF)
    ee2 = _proj(e, WE.T, 2000).reshape(NC * E * HALF)

    eout2, hatt2, w2 = _sc_edge(k2, q2, v2, ee2, src, dst)
    eout2 = eout2.reshape(NC * E, HALF)
    ssum2 = _sc_ssum(w2, src)

    y_h, s1h, q1h = _post_a_h(hatt2, ssum2, h, WOh.T,
                              bOh.reshape(1, D), 1000)
    f_h, s2h, q2h = _post_b(y_h, s1h, q1h, bn1h_g.reshape(1, D),
                            bn1h_b.reshape(1, D), W1h.T, b1h.reshape(1, 2 * D),
                            W2h.T, b2h.reshape(1, D), 1000)
    h_out = _post_c(f_h, s2h, q2h, bn2h_g.reshape(1, D),
                    bn2h_b.reshape(1, D), 1000)

    y_e, s1e, q1e = _post_a_e(eout2, e, WOe.T, bOe.reshape(1, D), 2000)
    f_e, s2e, q2e = _post_b(y_e, s1e, q1e, bn1e_g.reshape(1, D),
                            bn1e_b.reshape(1, D), W1e.T, b1e.reshape(1, 2 * D),
                            W2e.T, b2e.reshape(1, D), 2000)
    e_out = _post_c(f_e, s2e, q2e, bn2e_g.reshape(1, D),
                    bn2e_b.reshape(1, D), 2000)

    return (h_out, e_out)


# speedup vs baseline: 4.0070x; 1.1815x over previous
import functools
import math

import jax
import jax.numpy as jnp
from jax import lax
from jax.experimental import pallas as pl
from jax.experimental.pallas import tpu as pltpu
from jax.experimental.pallas import tpu_sc as plsc

N = 10000
E = 160000
D = 256
H = 8
DH = 32
HALF = D // 2

NC = 2
NS = 16
L = 16

ET = E // NS
EC = 80
NCHUNK = ET // EC
RPT = 624
NZ = 208
NREM = N - NS * RPT

_F32 = jnp.float32
_I32 = jnp.int32


def _sc_edge_body(k_hbm, q_hbm, v_hbm, ee_hbm, src_hbm, dst_hbm,
                  z128_hbm,
                  eout_hbm, hatt_hbm, w_hbm,
                  srcd_v, adjs_v, kq_v, q_v, t_v, w_v,
                  acc_sh, sem_i, sem_g, sem_o):
    s = lax.axis_index("s")
    c = lax.axis_index("c")

    row0 = s * RPT
    pltpu.sync_copy(z128_hbm.at[pl.ds(row0, RPT)], acc_sh.at[pl.ds(row0, RPT)])

    @pl.when(s < 2)
    def _zero_rem():
        r = NS * RPT + s * 8
        pltpu.sync_copy(z128_hbm.at[pl.ds(r, 8)], acc_sh.at[pl.ds(r, 8)])

    plsc.subcore_barrier()

    lanes = lax.iota(_I32, L)

    pltpu.sync_copy(src_hbm.at[pl.ds(s * ET, EC)], srcd_v.at[0, 0])
    pltpu.sync_copy(dst_hbm.at[pl.ds(s * ET, EC)], srcd_v.at[0, 1])

    @pl.loop(0, NCHUNK)
    def _chunk(ch):
        p = lax.rem(ch, 2)
        pn = 1 - p
        base = s * ET + ch * EC
        base_n = s * ET + jnp.minimum(ch + 1, NCHUNK - 1) * EC

        cpi1 = pltpu.async_copy(src_hbm.at[pl.ds(base_n, EC)],
                                srcd_v.at[pn, 0], sem_i)
        cpi2 = pltpu.async_copy(dst_hbm.at[pl.ds(base_n, EC)],
                                srcd_v.at[pn, 1], sem_i)

        off = c * N
        for g in range(EC // L):
            sl = pl.ds(g * L, L)
            adjs_v[sl] = srcd_v[p, 0, sl] + off
            srcd_v[p, 1, sl] = srcd_v[p, 1, sl] + off

        cp_k = pltpu.async_copy(k_hbm.at[adjs_v], kq_v, sem_g)
        cp_q = pltpu.async_copy(q_hbm.at[srcd_v.at[p, 1]], q_v, sem_g)
        cp_e = pltpu.async_copy(
            ee_hbm.at[pl.ds((c * E + base) * HALF, EC * HALF)], t_v, sem_g)
        cp_k.wait()
        cp_q.wait()
        cp_e.wait()

        @pl.loop(0, EC)
        def _tmul(ei):
            tbase = ei * HALF
            for j in range(HALF // L):
                tsl = pl.ds(tbase + j * L, L)
                t_v[tsl] = kq_v[ei, pl.ds(j * L, L)] * \
                    q_v[ei, pl.ds(j * L, L)] * t_v[tsl]

        cp_v = pltpu.async_copy(v_hbm.at[srcd_v.at[p, 1]], kq_v, sem_g)
        cp_out = pltpu.async_copy(
            t_v, eout_hbm.at[pl.ds((c * E + base) * HALF, EC * HALF)], sem_o)

        perms = [(lanes ^ k) for k in (1, 2, 4, 8)]

        @pl.loop(0, EC)
        def _score(ei):
            tbase = ei * HALF
            wrow = jnp.zeros((L,), _F32)
            for hh in range(H // 2):
                a = t_v[pl.ds(tbase + 2 * hh * L, L)] + \
                    t_v[pl.ds(tbase + (2 * hh + 1) * L, L)]
                for pp in perms:
                    a = a + a.at[pp].get(mode="promise_in_bounds")
                wrow = jnp.where(lanes == hh, jnp.exp(a), wrow)
            w_v[ei, :] = wrow

        cp_w = pltpu.async_copy(w_v, w_hbm.at[pl.ds(c * E + base, EC)], sem_o)
        cp_v.wait()

        @pl.loop(0, EC)
        def _wv(ei):
            wrow = w_v[ei, :]
            for hh in range(H // 2):
                bc = jnp.full((L,), wrow[hh], _F32)
                for j2 in range(DH // L):
                    sl = pl.ds(hh * DH + j2 * L, L)
                    kq_v[ei, sl] = kq_v[ei, sl] * bc

        pltpu.sync_copy(kq_v, acc_sh.at[srcd_v.at[p, 0]], add=True)
        cp_out.wait()
        cp_w.wait()
        cpi1.wait()
        cpi2.wait()

    plsc.subcore_barrier()

    pltpu.sync_copy(acc_sh.at[pl.ds(row0, RPT)],
                    hatt_hbm.at[pl.ds(c * N + row0, RPT)])

    @pl.when(s < 2)
    def _dump_rem():
        r = NS * RPT + s * 8
        pltpu.sync_copy(acc_sh.at[pl.ds(r, 8)],
                        hatt_hbm.at[pl.ds(c * N + r, 8)])


def _sc_edge(k2, q2, v2, ee2, src, dst):
    mesh = plsc.VectorSubcoreMesh(core_axis_name="c", subcore_axis_name="s")
    fn = pl.kernel(
        _sc_edge_body,
        out_type=[
            jax.ShapeDtypeStruct((NC * E * HALF,), _F32),
            jax.ShapeDtypeStruct((NC * N, HALF), _F32),
            jax.ShapeDtypeStruct((NC * E, L), _F32),
        ],
        mesh=mesh,
        scratch_types=[
            pltpu.VMEM((2, 2, EC), _I32),
            pltpu.VMEM((EC,), _I32),
            pltpu.VMEM((EC, HALF), _F32),
            pltpu.VMEM((EC, HALF), _F32),
            pltpu.VMEM((EC * HALF,), _F32),
            pltpu.VMEM((EC, L), _F32),
            pltpu.VMEM_SHARED((N, HALF), _F32),
            pltpu.SemaphoreType.DMA,
            pltpu.SemaphoreType.DMA,
            pltpu.SemaphoreType.DMA,
        ],
    )
    z128 = jnp.zeros((N, HALF), _F32)
    return fn(k2, q2, v2, ee2, src, dst, z128)


def _sc_ssum_body(w_hbm, src_hbm, z128_hbm, ssum_hbm,
                  src_v, w_v, x_v, acc_sh, sem_i, sem_s):
    s = lax.axis_index("s")
    c = lax.axis_index("c")

    row0 = s * RPT
    pltpu.sync_copy(z128_hbm.at[pl.ds(row0, RPT)], acc_sh.at[pl.ds(row0, RPT)])

    @pl.when(s < 2)
    def _zero_rem():
        r = NS * RPT + s * 8
        pltpu.sync_copy(z128_hbm.at[pl.ds(r, 8)], acc_sh.at[pl.ds(r, 8)])

    @pl.loop(0, EC)
    def _zero_x(ei):
        for p2 in range(2):
            for j in range(HALF // L):
                x_v[p2, ei, pl.ds(j * L, L)] = jnp.zeros((L,), _F32)

    plsc.subcore_barrier()

    pltpu.sync_copy(src_hbm.at[pl.ds(s * ET, EC)], src_v.at[0])
    pltpu.sync_copy(w_hbm.at[pl.ds(c * E + s * ET, EC)], w_v.at[0])

    @pl.loop(0, NCHUNK)
    def _chunk(ch):
        p = lax.rem(ch, 2)
        pn = 1 - p
        base_n = s * ET + jnp.minimum(ch + 1, NCHUNK - 1) * EC
        cpi = pltpu.async_copy(src_hbm.at[pl.ds(base_n, EC)],
                               src_v.at[pn], sem_i)
        cpw = pltpu.async_copy(w_hbm.at[pl.ds(c * E + base_n, EC)],
                               w_v.at[pn], sem_i)

        @pl.when(ch >= 2)
        def _drain():
            pltpu.make_async_copy(
                z128_hbm.at[pl.ds(0, EC)], x_v.at[p], sem_s).wait()

        @pl.loop(0, EC)
        def _expand(ei):
            x_v[p, ei, pl.ds(0, L)] = w_v[p, ei, :]

        pltpu.async_copy(x_v.at[p], acc_sh.at[src_v.at[p]], sem_s, add=True)
        cpi.wait()
        cpw.wait()

    for p2 in range(2):
        pltpu.make_async_copy(
            z128_hbm.at[pl.ds(0, EC)], x_v.at[p2], sem_s).wait()

    plsc.subcore_barrier()

    pltpu.sync_copy(acc_sh.at[pl.ds(row0, RPT)],
                    ssum_hbm.at[pl.ds(c * N + row0, RPT)])

    @pl.when(s < 2)
    def _dump_rem():
        r = NS * RPT + s * 8
        pltpu.sync_copy(acc_sh.at[pl.ds(r, 8)],
                        ssum_hbm.at[pl.ds(c * N + r, 8)])


def _sc_ssum(w2, src):
    mesh = plsc.VectorSubcoreMesh(core_axis_name="c", subcore_axis_name="s")
    fn = pl.kernel(
        _sc_ssum_body,
        out_type=[jax.ShapeDtypeStruct((NC * N, HALF), _F32)],
        mesh=mesh,
        scratch_types=[
            pltpu.VMEM((2, EC), _I32),
            pltpu.VMEM((2, EC, L), _F32),
            pltpu.VMEM((2, EC, HALF), _F32),
            pltpu.VMEM_SHARED((N, HALF), _F32),
            pltpu.SemaphoreType.DMA,
            pltpu.SemaphoreType.DMA,
        ],
    )
    z128 = jnp.zeros((N, HALF), _F32)
    return fn(w2, src, z128)[0]


def _proj_body(x_ref, w_ref, o_ref, nout):
    y = jnp.dot(x_ref[...], w_ref[...], preferred_element_type=_F32)
    for k in range(nout):
        o_ref[k, :, :] = y[:, k * HALF:(k + 1) * HALF]


def _proj(x, w, bn):
    rows, _ = x.shape
    nout = w.shape[1] // HALF
    grid = rows // bn
    return pl.pallas_call(
        functools.partial(_proj_body, nout=nout),
        grid=(grid,),
        in_specs=[
            pl.BlockSpec((bn, D), lambda i: (i, 0)),
            pl.BlockSpec((D, nout * HALF), lambda i: (0, 0)),
        ],
        out_specs=pl.BlockSpec((nout, bn, HALF), lambda i: (0, i, 0)),
        out_shape=jax.ShapeDtypeStruct((nout, rows, HALF), _F32),
    )(x, w)


def _stats_update(ref_sum, ref_sq, y, first):
    @pl.when(first)
    def _init():
        ref_sum[...] = jnp.zeros_like(ref_sum)
        ref_sq[...] = jnp.zeros_like(ref_sq)

    cs = jnp.sum(y, axis=0, keepdims=True)
    cq = jnp.sum(y * y, axis=0, keepdims=True)
    ref_sum[...] = ref_sum[...] + jnp.broadcast_to(cs, ref_sum.shape)
    ref_sq[...] = ref_sq[...] + jnp.broadcast_to(cq, ref_sq.shape)


def _post_a_h_body(a0, a1, s0, s1, xin, w_ref, b_ref,
                   y_ref, sum_ref, sq_ref):
    col = lax.broadcasted_iota(_I32, (L, HALF), 1) // DH
    row = lax.broadcasted_iota(_I32, (L, HALF), 0)
    bmat = (col == row).astype(_F32)
    s0v = s0[:, :L]
    s1v = s1[:, :L]
    r0 = jnp.where(s0v > 0, 1.0 / jnp.where(s0v > 0, s0v, 1.0), 0.0)
    r1 = jnp.where(s1v > 0, 1.0 / jnp.where(s1v > 0, s1v, 1.0), 0.0)
    att0 = a0[...] * jnp.dot(r0, bmat, preferred_element_type=_F32)
    att1 = a1[...] * jnp.dot(r1, bmat, preferred_element_type=_F32)
    att = jnp.concatenate([att0, att1], axis=1)
    y = jnp.dot(att, w_ref[...], preferred_element_type=_F32)
    y = y + b_ref[...] + xin[...]
    y_ref[...] = y
    _stats_update(sum_ref, sq_ref, y, pl.program_id(0) == 0)


def _post_a_e_body(a0, a1, xin, w_ref, b_ref, y_ref, sum_ref, sq_ref):
    att = jnp.concatenate([a0[...], a1[...]], axis=1)
    y = jnp.dot(att, w_ref[...], preferred_element_type=_F32)
    y = y + b_ref[...] + xin[...]
    y_ref[...] = y
    _stats_update(sum_ref, sq_ref, y, pl.program_id(0) == 0)


def _bn(y, sum_ref, sq_ref, g_ref, b_ref, rows):
    mean = sum_ref[0:1, :] * (1.0 / rows)
    var = sq_ref[0:1, :] * (1.0 / rows) - mean * mean
    inv = lax.rsqrt(var + 1e-5)
    return (y - mean) * inv * g_ref[...] + b_ref[...]


def _post_b_body(y_in, sum_ref, sq_ref, g_ref, b_ref, w1_ref, b1_ref,
                 w2_ref, b2_ref, f_ref, sum2_ref, sq2_ref, rows):
    z = _bn(y_in[...], sum_ref, sq_ref, g_ref, b_ref, rows)
    u = jnp.dot(z, w1_ref[...], preferred_element_type=_F32) + b1_ref[...]
    u = jnp.maximum(u, 0.0)
    f = jnp.dot(u, w2_ref[...], preferred_element_type=_F32) + b2_ref[...] + z
    f_ref[...] = f
    _stats_update(sum2_ref, sq2_ref, f, pl.program_id(0) == 0)


def _post_c_body(f_in, sum_ref, sq_ref, g_ref, b_ref, o_ref, rows):
    o_ref[...] = _bn(f_in[...], sum_ref, sq_ref, g_ref, b_ref, rows)


def _full_spec(shape):
    return pl.BlockSpec(shape, lambda i: tuple(0 for _ in shape))


def _post_a_h(hatt2, ssum2, h_in, wo, bo, bn):
    grid = N // bn
    blk = lambda r0: pl.BlockSpec((bn, HALF), lambda i, r0=r0: (r0 + i, 0))
    blks = blk
    return pl.pallas_call(
        _post_a_h_body,
        grid=(grid,),
        in_specs=[
            blk(0), blk(N // bn), blks(0), blks(N // bn),
            pl.BlockSpec((bn, D), lambda i: (i, 0)),
            _full_spec((D, D)), _full_spec((1, D)),
        ],
        out_specs=[
            pl.BlockSpec((bn, D), lambda i: (i, 0)),
            _full_spec((8, D)), _full_spec((8, D)),
        ],
        out_shape=[
            jax.ShapeDtypeStruct((N, D), _F32),
            jax.ShapeDtypeStruct((8, D), _F32),
            jax.ShapeDtypeStruct((8, D), _F32),
        ],
    )(hatt2, hatt2, ssum2, ssum2, h_in, wo, bo)


def _post_a_e(eout2, e_in, wo, bo, bn):
    grid = E // bn
    blk = lambda r0: pl.BlockSpec((bn, HALF), lambda i, r0=r0: (r0 + i, 0))
    return pl.pallas_call(
        _post_a_e_body,
        grid=(grid,),
        in_specs=[
            blk(0), blk(E // bn),
            pl.BlockSpec((bn, D), lambda i: (i, 0)),
            _full_spec((D, D)), _full_spec((1, D)),
        ],
        out_specs=[
            pl.BlockSpec((bn, D), lambda i: (i, 0)),
            _full_spec((8, D)), _full_spec((8, D)),
        ],
        out_shape=[
            jax.ShapeDtypeStruct((E, D), _F32),
            jax.ShapeDtypeStruct((8, D), _F32),
            jax.ShapeDtypeStruct((8, D), _F32),
        ],
    )(eout2, eout2, e_in, wo, bo)


def _post_b(y, sums, sqs, g, b, w1, b1, w2, b2, bn):
    rows = y.shape[0]
    grid = rows // bn
    return pl.pallas_call(
        functools.partial(_post_b_body, rows=float(rows)),
        grid=(grid,),
        in_specs=[
            pl.BlockSpec((bn, D), lambda i: (i, 0)),
            _full_spec((8, D)), _full_spec((8, D)),
            _full_spec((1, D)), _full_spec((1, D)),
            _full_spec((D, 2 * D)), _full_spec((1, 2 * D)),
            _full_spec((2 * D, D)), _full_spec((1, D)),
        ],
        out_specs=[
            pl.BlockSpec((bn, D), lambda i: (i, 0)),
            _full_spec((8, D)), _full_spec((8, D)),
        ],
        out_shape=[
            jax.ShapeDtypeStruct((rows, D), _F32),
            jax.ShapeDtypeStruct((8, D), _F32),
            jax.ShapeDtypeStruct((8, D), _F32),
        ],
    )(y, sums, sqs, g, b, w1, b1, w2, b2)


def _post_c(f, sums, sqs, g, b, bn):
    rows = f.shape[0]
    grid = rows // bn
    return pl.pallas_call(
        functools.partial(_post_c_body, rows=float(rows)),
        grid=(grid,),
        in_specs=[
            pl.BlockSpec((bn, D), lambda i: (i, 0)),
            _full_spec((8, D)), _full_spec((8, D)),
            _full_spec((1, D)), _full_spec((1, D)),
        ],
        out_specs=pl.BlockSpec((bn, D), lambda i: (i, 0)),
        out_shape=jax.ShapeDtypeStruct((rows, D), _F32),
    )(f, sums, sqs, g, b)


def kernel(h, e, edge_index, WQ, WK, WV, WE, WOh, bOh, WOe, bOe,
           bn1h_g, bn1h_b, bn1e_g, bn1e_b, W1h, b1h, W2h, b2h,
           W1e, b1e, W2e, b2e, bn2h_g, bn2h_b, bn2e_g, bn2e_b):
    src = edge_index[0].astype(_I32)
    dst = edge_index[1].astype(_I32)
    scale = 1.0 / math.sqrt(DH)

    wq = WQ.T
    wk = WK.T * scale
    wv = WV.T
    wkqv = jnp.concatenate([wk, wq, wv], axis=1)

    kqv = _proj(h, wkqv, 1000)
    k2 = kqv[0:2].reshape(NC * N, HALF)
    q2 = kqv[2:4].reshape(NC * N, HALF)
    v2 = kqv[4:6].reshape(NC * N, HALF)
    ee2 = _proj(e, WE.T, 2000).reshape(NC * E * HALF)

    eout2, hatt2, w2 = _sc_edge(k2, q2, v2, ee2, src, dst)
    eout2 = eout2.reshape(NC * E, HALF)
    ssum2 = _sc_ssum(w2, src)

    y_h, s1h, q1h = _post_a_h(hatt2, ssum2, h, WOh.T,
                              bOh.reshape(1, D), 1000)
    f_h, s2h, q2h = _post_b(y_h, s1h, q1h, bn1h_g.reshape(1, D),
                            bn1h_b.reshape(1, D), W1h.T, b1h.reshape(1, 2 * D),
                            W2h.T, b2h.reshape(1, D), 1000)
    h_out = _post_c(f_h, s2h, q2h, bn2h_g.reshape(1, D),
                    bn2h_b.reshape(1, D), 1000)

    y_e, s1e, q1e = _post_a_e(eout2, e, WOe.T, bOe.reshape(1, D), 2000)
    f_e, s2e, q2e = _post_b(y_e, s1e, q1e, bn1e_g.reshape(1, D),
                            bn1e_b.reshape(1, D), W1e.T, b1e.reshape(1, 2 * D),
                            W2e.T, b2e.reshape(1, D), 2000)
    e_out = _post_c(f_e, s2e, q2e, bn2e_g.reshape(1, D),
                    bn2e_b.reshape(1, D), 2000)

    return (h_out, e_out)


# speedup vs baseline: 4.0168x; 1.0024x over previous
import functools
import math

import jax
import jax.numpy as jnp
from jax import lax
from jax.experimental import pallas as pl
from jax.experimental.pallas import tpu as pltpu
from jax.experimental.pallas import tpu_sc as plsc

N = 10000
E = 160000
D = 256
H = 8
DH = 32
HALF = D // 2

NC = 2
NS = 16
L = 16

ET = E // NS
EC = 80
NCHUNK = ET // EC
RPT = 624
NZ = 208
NREM = N - NS * RPT

_F32 = jnp.float32
_I32 = jnp.int32


def _sc_edge_body(k_hbm, q_hbm, v_hbm, ee_hbm, src_hbm, dst_hbm,
                  z128_hbm,
                  eout_hbm, hatt_hbm, w_hbm,
                  srcd_v, adjs_v, kq_v, q_v, t_v, w_v,
                  acc_sh, sem_i, sem_g, sem_o, sem_s):
    s = lax.axis_index("s")
    c = lax.axis_index("c")

    row0 = s * RPT
    pltpu.sync_copy(z128_hbm.at[pl.ds(row0, RPT)], acc_sh.at[pl.ds(row0, RPT)])

    @pl.when(s < 2)
    def _zero_rem():
        r = NS * RPT + s * 8
        pltpu.sync_copy(z128_hbm.at[pl.ds(r, 8)], acc_sh.at[pl.ds(r, 8)])

    plsc.subcore_barrier()

    lanes = lax.iota(_I32, L)

    pltpu.sync_copy(src_hbm.at[pl.ds(s * ET, EC)], srcd_v.at[0, 0])
    pltpu.sync_copy(dst_hbm.at[pl.ds(s * ET, EC)], srcd_v.at[0, 1])

    @pl.loop(0, NCHUNK)
    def _chunk(ch):
        p = lax.rem(ch, 3)
        pn = lax.rem(ch + 1, 3)
        base = s * ET + ch * EC
        base_n = s * ET + jnp.minimum(ch + 1, NCHUNK - 1) * EC

        cpi1 = pltpu.async_copy(src_hbm.at[pl.ds(base_n, EC)],
                                srcd_v.at[pn, 0], sem_i)
        cpi2 = pltpu.async_copy(dst_hbm.at[pl.ds(base_n, EC)],
                                srcd_v.at[pn, 1], sem_i)

        off = c * N
        for g in range(EC // L):
            sl = pl.ds(g * L, L)
            adjs_v[sl] = srcd_v[p, 0, sl] + off
            srcd_v[p, 1, sl] = srcd_v[p, 1, sl] + off

        @pl.when(ch >= 1)
        def _drain_sct():
            pltpu.make_async_copy(
                z128_hbm.at[pl.ds(0, EC)], kq_v, sem_s).wait()

        cp_k = pltpu.async_copy(k_hbm.at[adjs_v], kq_v, sem_g)
        cp_q = pltpu.async_copy(q_hbm.at[srcd_v.at[p, 1]], q_v, sem_g)
        cp_e = pltpu.async_copy(
            ee_hbm.at[pl.ds((c * E + base) * HALF, EC * HALF)], t_v, sem_g)
        cp_k.wait()
        cp_q.wait()
        cp_e.wait()

        @pl.loop(0, EC)
        def _tmul(ei):
            tbase = ei * HALF
            for j in range(HALF // L):
                tsl = pl.ds(tbase + j * L, L)
                t_v[tsl] = kq_v[ei, pl.ds(j * L, L)] * \
                    q_v[ei, pl.ds(j * L, L)] * t_v[tsl]

        cp_v = pltpu.async_copy(v_hbm.at[srcd_v.at[p, 1]], kq_v, sem_g)
        cp_out = pltpu.async_copy(
            t_v, eout_hbm.at[pl.ds((c * E + base) * HALF, EC * HALF)], sem_o)

        perm1 = lanes ^ 1
        perm2 = lanes ^ 2
        perm4 = lanes ^ 4
        perm8 = lanes ^ 8
        lm4 = lanes & 3

        @pl.loop(0, EC)
        def _score(ei):
            tbase = ei * HALF
            a = []
            for hh in range(H // 2):
                x = t_v[pl.ds(tbase + 2 * hh * L, L)] + \
                    t_v[pl.ds(tbase + (2 * hh + 1) * L, L)]
                x = x + x.at[perm1].get(mode="promise_in_bounds")
                x = x + x.at[perm2].get(mode="promise_in_bounds")
                a.append(x)
            comb = jnp.where(lm4 == 0, a[0],
                             jnp.where(lm4 == 1, a[1],
                                       jnp.where(lm4 == 2, a[2], a[3])))
            comb = comb + comb.at[perm4].get(mode="promise_in_bounds")
            comb = comb + comb.at[perm8].get(mode="promise_in_bounds")
            w_v[ei, :] = jnp.exp(comb)

        cp_w = pltpu.async_copy(w_v, w_hbm.at[pl.ds(c * E + base, EC)], sem_o)
        cp_v.wait()

        @pl.loop(0, EC)
        def _wv(ei):
            wrow = w_v[ei, :]
            for hh in range(H // 2):
                bc = jnp.full((L,), wrow[hh], _F32)
                for j2 in range(DH // L):
                    sl = pl.ds(hh * DH + j2 * L, L)
                    kq_v[ei, sl] = kq_v[ei, sl] * bc

        pltpu.async_copy(kq_v, acc_sh.at[srcd_v.at[p, 0]], sem_s, add=True)
        cp_out.wait()
        cp_w.wait()
        cpi1.wait()
        cpi2.wait()

    pltpu.make_async_copy(z128_hbm.at[pl.ds(0, EC)], kq_v, sem_s).wait()

    plsc.subcore_barrier()

    pltpu.sync_copy(acc_sh.at[pl.ds(row0, RPT)],
                    hatt_hbm.at[pl.ds(c * N + row0, RPT)])

    @pl.when(s < 2)
    def _dump_rem():
        r = NS * RPT + s * 8
        pltpu.sync_copy(acc_sh.at[pl.ds(r, 8)],
                        hatt_hbm.at[pl.ds(c * N + r, 8)])


def _sc_edge(k2, q2, v2, ee2, src, dst):
    mesh = plsc.VectorSubcoreMesh(core_axis_name="c", subcore_axis_name="s")
    fn = pl.kernel(
        _sc_edge_body,
        out_type=[
            jax.ShapeDtypeStruct((NC * E * HALF,), _F32),
            jax.ShapeDtypeStruct((NC * N, HALF), _F32),
            jax.ShapeDtypeStruct((NC * E, L), _F32),
        ],
        mesh=mesh,
        scratch_types=[
            pltpu.VMEM((3, 2, EC), _I32),
            pltpu.VMEM((EC,), _I32),
            pltpu.VMEM((EC, HALF), _F32),
            pltpu.VMEM((EC, HALF), _F32),
            pltpu.VMEM((EC * HALF,), _F32),
            pltpu.VMEM((EC, L), _F32),
            pltpu.VMEM_SHARED((N, HALF), _F32),
            pltpu.SemaphoreType.DMA,
            pltpu.SemaphoreType.DMA,
            pltpu.SemaphoreType.DMA,
            pltpu.SemaphoreType.DMA,
        ],
    )
    z128 = jnp.zeros((N, HALF), _F32)
    return fn(k2, q2, v2, ee2, src, dst, z128)


def _sc_ssum_body(w_hbm, src_hbm, z128_hbm, ssum_hbm,
                  src_v, w_v, x_v, acc_sh, sem_i, sem_s):
    s = lax.axis_index("s")
    c = lax.axis_index("c")

    row0 = s * RPT
    pltpu.sync_copy(z128_hbm.at[pl.ds(row0, RPT)], acc_sh.at[pl.ds(row0, RPT)])

    @pl.when(s < 2)
    def _zero_rem():
        r = NS * RPT + s * 8
        pltpu.sync_copy(z128_hbm.at[pl.ds(r, 8)], acc_sh.at[pl.ds(r, 8)])

    @pl.loop(0, EC)
    def _zero_x(ei):
        for p2 in range(2):
            for j in range(HALF // L):
                x_v[p2, ei, pl.ds(j * L, L)] = jnp.zeros((L,), _F32)

    plsc.subcore_barrier()

    pltpu.sync_copy(src_hbm.at[pl.ds(s * ET, EC)], src_v.at[0])
    pltpu.sync_copy(w_hbm.at[pl.ds(c * E + s * ET, EC)], w_v.at[0])

    @pl.loop(0, NCHUNK)
    def _chunk(ch):
        p = lax.rem(ch, 2)
        pn = 1 - p
        base_n = s * ET + jnp.minimum(ch + 1, NCHUNK - 1) * EC
        cpi = pltpu.async_copy(src_hbm.at[pl.ds(base_n, EC)],
                               src_v.at[pn], sem_i)
        cpw = pltpu.async_copy(w_hbm.at[pl.ds(c * E + base_n, EC)],
                               w_v.at[pn], sem_i)

        @pl.when(ch >= 2)
        def _drain():
            pltpu.make_async_copy(
                z128_hbm.at[pl.ds(0, EC)], x_v.at[p], sem_s).wait()

        @pl.loop(0, EC)
        def _expand(ei):
            x_v[p, ei, pl.ds(0, L)] = w_v[p, ei, :]

        pltpu.async_copy(x_v.at[p], acc_sh.at[src_v.at[p]], sem_s, add=True)
        cpi.wait()
        cpw.wait()

    for p2 in range(2):
        pltpu.make_async_copy(
            z128_hbm.at[pl.ds(0, EC)], x_v.at[p2], sem_s).wait()

    plsc.subcore_barrier()

    pltpu.sync_copy(acc_sh.at[pl.ds(row0, RPT)],
                    ssum_hbm.at[pl.ds(c * N + row0, RPT)])

    @pl.when(s < 2)
    def _dump_rem():
        r = NS * RPT + s * 8
        pltpu.sync_copy(acc_sh.at[pl.ds(r, 8)],
                        ssum_hbm.at[pl.ds(c * N + r, 8)])


def _sc_ssum(w2, src):
    mesh = plsc.VectorSubcoreMesh(core_axis_name="c", subcore_axis_name="s")
    fn = pl.kernel(
        _sc_ssum_body,
        out_type=[jax.ShapeDtypeStruct((NC * N, HALF), _F32)],
        mesh=mesh,
        scratch_types=[
            pltpu.VMEM((2, EC), _I32),
            pltpu.VMEM((2, EC, L), _F32),
            pltpu.VMEM((2, EC, HALF), _F32),
            pltpu.VMEM_SHARED((N, HALF), _F32),
            pltpu.SemaphoreType.DMA,
            pltpu.SemaphoreType.DMA,
        ],
    )
    z128 = jnp.zeros((N, HALF), _F32)
    return fn(w2, src, z128)[0]


def _proj_body(x_ref, w_ref, o_ref, nout):
    y = jnp.dot(x_ref[...], w_ref[...], preferred_element_type=_F32)
    for k in range(nout):
        o_ref[k, :, :] = y[:, k * HALF:(k + 1) * HALF]


def _proj(x, w, bn):
    rows, _ = x.shape
    nout = w.shape[1] // HALF
    grid = rows // bn
    return pl.pallas_call(
        functools.partial(_proj_body, nout=nout),
        grid=(grid,),
        in_specs=[
            pl.BlockSpec((bn, D), lambda i: (i, 0)),
            pl.BlockSpec((D, nout * HALF), lambda i: (0, 0)),
        ],
        out_specs=pl.BlockSpec((nout, bn, HALF), lambda i: (0, i, 0)),
        out_shape=jax.ShapeDtypeStruct((nout, rows, HALF), _F32),
    )(x, w)


def _stats_update(ref_sum, ref_sq, y, first):
    @pl.when(first)
    def _init():
        ref_sum[...] = jnp.zeros_like(ref_sum)
        ref_sq[...] = jnp.zeros_like(ref_sq)

    cs = jnp.sum(y, axis=0, keepdims=True)
    cq = jnp.sum(y * y, axis=0, keepdims=True)
    ref_sum[...] = ref_sum[...] + jnp.broadcast_to(cs, ref_sum.shape)
    ref_sq[...] = ref_sq[...] + jnp.broadcast_to(cq, ref_sq.shape)


def _post_a_h_body(a0, a1, s0, s1, xin, w_ref, b_ref,
                   y_ref, sum_ref, sq_ref):
    col = lax.broadcasted_iota(_I32, (L, HALF), 1) // DH
    row = lax.broadcasted_iota(_I32, (L, HALF), 0)
    bmat = (col == row).astype(_F32)
    s0v = s0[:, :L]
    s1v = s1[:, :L]
    r0 = jnp.where(s0v > 0, 1.0 / jnp.where(s0v > 0, s0v, 1.0), 0.0)
    r1 = jnp.where(s1v > 0, 1.0 / jnp.where(s1v > 0, s1v, 1.0), 0.0)
    att0 = a0[...] * jnp.dot(r0, bmat, preferred_element_type=_F32)
    att1 = a1[...] * jnp.dot(r1, bmat, preferred_element_type=_F32)
    att = jnp.concatenate([att0, att1], axis=1)
    y = jnp.dot(att, w_ref[...], preferred_element_type=_F32)
    y = y + b_ref[...] + xin[...]
    y_ref[...] = y
    _stats_update(sum_ref, sq_ref, y, pl.program_id(0) == 0)


def _post_a_e_body(a0, a1, xin, w_ref, b_ref, y_ref, sum_ref, sq_ref):
    att = jnp.concatenate([a0[...], a1[...]], axis=1)
    y = jnp.dot(att, w_ref[...], preferred_element_type=_F32)
    y = y + b_ref[...] + xin[...]
    y_ref[...] = y
    _stats_update(sum_ref, sq_ref, y, pl.program_id(0) == 0)


def _bn(y, sum_ref, sq_ref, g_ref, b_ref, rows):
    mean = sum_ref[0:1, :] * (1.0 / rows)
    var = sq_ref[0:1, :] * (1.0 / rows) - mean * mean
    inv = lax.rsqrt(var + 1e-5)
    return (y - mean) * inv * g_ref[...] + b_ref[...]


def _post_b_body(y_in, sum_ref, sq_ref, g_ref, b_ref, w1_ref, b1_ref,
                 w2_ref, b2_ref, f_ref, sum2_ref, sq2_ref, rows):
    z = _bn(y_in[...], sum_ref, sq_ref, g_ref, b_ref, rows)
    u = jnp.dot(z, w1_ref[...], preferred_element_type=_F32) + b1_ref[...]
    u = jnp.maximum(u, 0.0)
    f = jnp.dot(u, w2_ref[...], preferred_element_type=_F32) + b2_ref[...] + z
    f_ref[...] = f
    _stats_update(sum2_ref, sq2_ref, f, pl.program_id(0) == 0)


def _post_c_body(f_in, sum_ref, sq_ref, g_ref, b_ref, o_ref, rows):
    o_ref[...] = _bn(f_in[...], sum_ref, sq_ref, g_ref, b_ref, rows)


def _full_spec(shape):
    return pl.BlockSpec(shape, lambda i: tuple(0 for _ in shape))


def _post_a_h(hatt2, ssum2, h_in, wo, bo, bn):
    grid = N // bn
    blk = lambda r0: pl.BlockSpec((bn, HALF), lambda i, r0=r0: (r0 + i, 0))
    blks = blk
    return pl.pallas_call(
        _post_a_h_body,
        grid=(grid,),
        in_specs=[
            blk(0), blk(N // bn), blks(0), blks(N // bn),
            pl.BlockSpec((bn, D), lambda i: (i, 0)),
            _full_spec((D, D)), _full_spec((1, D)),
        ],
        out_specs=[
            pl.BlockSpec((bn, D), lambda i: (i, 0)),
            _full_spec((8, D)), _full_spec((8, D)),
        ],
        out_shape=[
            jax.ShapeDtypeStruct((N, D), _F32),
            jax.ShapeDtypeStruct((8, D), _F32),
            jax.ShapeDtypeStruct((8, D), _F32),
        ],
    )(hatt2, hatt2, ssum2, ssum2, h_in, wo, bo)


def _post_a_e(eout2, e_in, wo, bo, bn):
    grid = E // bn
    blk = lambda r0: pl.BlockSpec((bn, HALF), lambda i, r0=r0: (r0 + i, 0))
    return pl.pallas_call(
        _post_a_e_body,
        grid=(grid,),
        in_specs=[
            blk(0), blk(E // bn),
            pl.BlockSpec((bn, D), lambda i: (i, 0)),
            _full_spec((D, D)), _full_spec((1, D)),
        ],
        out_specs=[
            pl.BlockSpec((bn, D), lambda i: (i, 0)),
            _full_spec((8, D)), _full_spec((8, D)),
        ],
        out_shape=[
            jax.ShapeDtypeStruct((E, D), _F32),
            jax.ShapeDtypeStruct((8, D), _F32),
            jax.ShapeDtypeStruct((8, D), _F32),
        ],
    )(eout2, eout2, e_in, wo, bo)


def _post_b(y, sums, sqs, g, b, w1, b1, w2, b2, bn):
    rows = y.shape[0]
    grid = rows // bn
    return pl.pallas_call(
        functools.partial(_post_b_body, rows=float(rows)),
        grid=(grid,),
        in_specs=[
            pl.BlockSpec((bn, D), lambda i: (i, 0)),
            _full_spec((8, D)), _full_spec((8, D)),
            _full_spec((1, D)), _full_spec((1, D)),
            _full_spec((D, 2 * D)), _full_spec((1, 2 * D)),
            _full_spec((2 * D, D)), _full_spec((1, D)),
        ],
        out_specs=[
            pl.BlockSpec((bn, D), lambda i: (i, 0)),
            _full_spec((8, D)), _full_spec((8, D)),
        ],
        out_shape=[
            jax.ShapeDtypeStruct((rows, D), _F32),
            jax.ShapeDtypeStruct((8, D), _F32),
            jax.ShapeDtypeStruct((8, D), _F32),
        ],
    )(y, sums, sqs, g, b, w1, b1, w2, b2)


def _post_c(f, sums, sqs, g, b, bn):
    rows = f.shape[0]
    grid = rows // bn
    return pl.pallas_call(
        functools.partial(_post_c_body, rows=float(rows)),
        grid=(grid,),
        in_specs=[
            pl.BlockSpec((bn, D), lambda i: (i, 0)),
            _full_spec((8, D)), _full_spec((8, D)),
            _full_spec((1, D)), _full_spec((1, D)),
        ],
        out_specs=pl.BlockSpec((bn, D), lambda i: (i, 0)),
        out_shape=jax.ShapeDtypeStruct((rows, D), _F32),
    )(f, sums, sqs, g, b)


def kernel(h, e, edge_index, WQ, WK, WV, WE, WOh, bOh, WOe, bOe,
           bn1h_g, bn1h_b, bn1e_g, bn1e_b, W1h, b1h, W2h, b2h,
           W1e, b1e, W2e, b2e, bn2h_g, bn2h_b, bn2e_g, bn2e_b):
    src = edge_index[0].astype(_I32)
    dst = edge_index[1].astype(_I32)
    scale = 1.0 / math.sqrt(DH)

    wq = WQ.T
    wk = WK.T * scale
    wv = WV.T
    wkqv = jnp.concatenate([wk, wq, wv], axis=1)

    kqv = _proj(h, wkqv, 1000)
    k2 = kqv[0:2].reshape(NC * N, HALF)
    q2 = kqv[2:4].reshape(NC * N, HALF)
    v2 = kqv[4:6].reshape(NC * N, HALF)
    ee2 = _proj(e, WE.T, 2000).reshape(NC * E * HALF)

    eout2, hatt2, w2 = _sc_edge(k2, q2, v2, ee2, src, dst)
    eout2 = eout2.reshape(NC * E, HALF)
    ssum2 = _sc_ssum(w2, src)

    y_h, s1h, q1h = _post_a_h(hatt2, ssum2, h, WOh.T,
                              bOh.reshape(1, D), 1000)
    f_h, s2h, q2h = _post_b(y_h, s1h, q1h, bn1h_g.reshape(1, D),
                            bn1h_b.reshape(1, D), W1h.T, b1h.reshape(1, 2 * D),
                            W2h.T, b2h.reshape(1, D), 1000)
    h_out = _post_c(f_h, s2h, q2h, bn2h_g.reshape(1, D),
                    bn2h_b.reshape(1, D), 1000)

    y_e, s1e, q1e = _post_a_e(eout2, e, WOe.T, bOe.reshape(1, D), 2000)
    f_e, s2e, q2e = _post_b(y_e, s1e, q1e, bn1e_g.reshape(1, D),
                            bn1e_b.reshape(1, D), W1e.T, b1e.reshape(1, 2 * D),
                            W2e.T, b2e.reshape(1, D), 2000)
    e_out = _post_c(f_e, s2e, q2e, bn2e_g.reshape(1, D),
                    bn2e_b.reshape(1, D), 2000)

    return (h_out, e_out)


# speedup vs baseline: 4.0210x; 1.0011x over previous
import functools
import math

import jax
import jax.numpy as jnp
from jax import lax
from jax.experimental import pallas as pl
from jax.experimental.pallas import tpu as pltpu
from jax.experimental.pallas import tpu_sc as plsc

N = 10000
E = 160000
D = 256
H = 8
DH = 32
HALF = D // 2

NC = 2
NS = 16
L = 16

ET = E // NS
EC = 80
NCHUNK = ET // EC
RPT = 624
NZ = 208
NREM = N - NS * RPT

_F32 = jnp.float32
_I32 = jnp.int32


def _sc_edge_body(k_hbm, q_hbm, v_hbm, ee_hbm, src_hbm, dst_hbm,
                  z128_hbm,
                  eout_hbm, hatt_hbm, w_hbm,
                  srcd_v, adjs_v, kq_v, q_v, t_v, w_v,
                  acc_sh, sem_i, sem_g, sem_o, sem_s):
    s = lax.axis_index("s")
    c = lax.axis_index("c")

    row0 = s * RPT
    pltpu.sync_copy(z128_hbm.at[pl.ds(row0, RPT)], acc_sh.at[pl.ds(row0, RPT)])

    @pl.when(s < 2)
    def _zero_rem():
        r = NS * RPT + s * 8
        pltpu.sync_copy(z128_hbm.at[pl.ds(r, 8)], acc_sh.at[pl.ds(r, 8)])

    plsc.subcore_barrier()

    lanes = lax.iota(_I32, L)

    pltpu.sync_copy(src_hbm.at[pl.ds(s * ET, EC)], srcd_v.at[0, 0])
    pltpu.sync_copy(dst_hbm.at[pl.ds(s * ET, EC)], srcd_v.at[0, 1])

    @pl.loop(0, NCHUNK)
    def _chunk(ch):
        p = lax.rem(ch, 3)
        pn = lax.rem(ch + 1, 3)
        base = s * ET + ch * EC
        base_n = s * ET + jnp.minimum(ch + 1, NCHUNK - 1) * EC

        cpi1 = pltpu.async_copy(src_hbm.at[pl.ds(base_n, EC)],
                                srcd_v.at[pn, 0], sem_i)
        cpi2 = pltpu.async_copy(dst_hbm.at[pl.ds(base_n, EC)],
                                srcd_v.at[pn, 1], sem_i)

        off = c * N
        for g in range(EC // L):
            sl = pl.ds(g * L, L)
            adjs_v[sl] = srcd_v[p, 0, sl] + off
            srcd_v[p, 1, sl] = srcd_v[p, 1, sl] + off

        @pl.when(ch >= 1)
        def _drain_sct():
            pltpu.make_async_copy(
                z128_hbm.at[pl.ds(0, EC)], kq_v, sem_s).wait()

        cp_k = pltpu.async_copy(k_hbm.at[adjs_v], kq_v, sem_g)
        cp_q = pltpu.async_copy(q_hbm.at[srcd_v.at[p, 1]], q_v, sem_g)
        cp_e = pltpu.async_copy(
            ee_hbm.at[pl.ds((c * E + base) * HALF, EC * HALF)], t_v, sem_g)
        cp_k.wait()
        cp_q.wait()
        cp_e.wait()

        @pl.loop(0, EC)
        def _tmul(ei):
            tbase = ei * HALF
            for j in range(HALF // L):
                tsl = pl.ds(tbase + j * L, L)
                t_v[tsl] = kq_v[ei, pl.ds(j * L, L)] * \
                    q_v[ei, pl.ds(j * L, L)] * t_v[tsl]

        cp_v = pltpu.async_copy(v_hbm.at[srcd_v.at[p, 1]], kq_v, sem_g)
        cp_out = pltpu.async_copy(
            t_v, eout_hbm.at[pl.ds((c * E + base) * HALF, EC * HALF)], sem_o)

        perm1 = lanes ^ 1
        perm2 = lanes ^ 2
        perm4 = lanes ^ 4
        perm8 = lanes ^ 8
        lm4 = lanes & 3

        @pl.loop(0, EC)
        def _score(ei):
            tbase = ei * HALF
            a = []
            for hh in range(H // 2):
                x = t_v[pl.ds(tbase + 2 * hh * L, L)] + \
                    t_v[pl.ds(tbase + (2 * hh + 1) * L, L)]
                x = x + x.at[perm1].get(mode="promise_in_bounds")
                x = x + x.at[perm2].get(mode="promise_in_bounds")
                a.append(x)
            comb = jnp.where(lm4 == 0, a[0],
                             jnp.where(lm4 == 1, a[1],
                                       jnp.where(lm4 == 2, a[2], a[3])))
            comb = comb + comb.at[perm4].get(mode="promise_in_bounds")
            comb = comb + comb.at[perm8].get(mode="promise_in_bounds")
            w_v[ei, :] = jnp.exp(comb)

        cp_w = pltpu.async_copy(w_v, w_hbm.at[pl.ds(c * E + base, EC)], sem_o)
        cp_v.wait()

        @pl.loop(0, EC)
        def _wv(ei):
            wrow = w_v[ei, :]
            for hh in range(H // 2):
                bc = jnp.full((L,), wrow[hh], _F32)
                for j2 in range(DH // L):
                    sl = pl.ds(hh * DH + j2 * L, L)
                    kq_v[ei, sl] = kq_v[ei, sl] * bc

        pltpu.async_copy(kq_v, acc_sh.at[srcd_v.at[p, 0]], sem_s, add=True)
        cp_out.wait()
        cp_w.wait()
        cpi1.wait()
        cpi2.wait()

    pltpu.make_async_copy(z128_hbm.at[pl.ds(0, EC)], kq_v, sem_s).wait()

    plsc.subcore_barrier()

    pltpu.sync_copy(acc_sh.at[pl.ds(row0, RPT)],
                    hatt_hbm.at[pl.ds(c * N + row0, RPT)])

    @pl.when(s < 2)
    def _dump_rem():
        r = NS * RPT + s * 8
        pltpu.sync_copy(acc_sh.at[pl.ds(r, 8)],
                        hatt_hbm.at[pl.ds(c * N + r, 8)])


def _sc_edge(k2, q2, v2, ee2, src, dst):
    mesh = plsc.VectorSubcoreMesh(core_axis_name="c", subcore_axis_name="s")
    fn = pl.kernel(
        _sc_edge_body,
        out_type=[
            jax.ShapeDtypeStruct((NC * E * HALF,), _F32),
            jax.ShapeDtypeStruct((NC * N, HALF), _F32),
            jax.ShapeDtypeStruct((NC * E, L), _F32),
        ],
        mesh=mesh,
        scratch_types=[
            pltpu.VMEM((3, 2, EC), _I32),
            pltpu.VMEM((EC,), _I32),
            pltpu.VMEM((EC, HALF), _F32),
            pltpu.VMEM((EC, HALF), _F32),
            pltpu.VMEM((EC * HALF,), _F32),
            pltpu.VMEM((EC, L), _F32),
            pltpu.VMEM_SHARED((N, HALF), _F32),
            pltpu.SemaphoreType.DMA,
            pltpu.SemaphoreType.DMA,
            pltpu.SemaphoreType.DMA,
            pltpu.SemaphoreType.DMA,
        ],
    )
    z128 = jnp.zeros((N, HALF), _F32)
    return fn(k2, q2, v2, ee2, src, dst, z128)


def _sc_ssum_body(w_hbm, src_hbm, z128_hbm, ssum_hbm,
                  src_v, w_v, x_v, acc_sh, sem_i, sem_s):
    s = lax.axis_index("s")
    c = lax.axis_index("c")

    row0 = s * RPT
    pltpu.sync_copy(z128_hbm.at[pl.ds(row0, RPT)], acc_sh.at[pl.ds(row0, RPT)])

    @pl.when(s < 2)
    def _zero_rem():
        r = NS * RPT + s * 8
        pltpu.sync_copy(z128_hbm.at[pl.ds(r, 8)], acc_sh.at[pl.ds(r, 8)])

    @pl.loop(0, EC)
    def _zero_x(ei):
        for p2 in range(2):
            for j in range(HALF // L):
                x_v[p2, ei, pl.ds(j * L, L)] = jnp.zeros((L,), _F32)

    plsc.subcore_barrier()

    pltpu.sync_copy(src_hbm.at[pl.ds(s * ET, EC)], src_v.at[0])
    pltpu.sync_copy(w_hbm.at[pl.ds(c * E + s * ET, EC)], w_v.at[0])

    @pl.loop(0, NCHUNK)
    def _chunk(ch):
        p = lax.rem(ch, 2)
        pn = 1 - p
        base_n = s * ET + jnp.minimum(ch + 1, NCHUNK - 1) * EC
        cpi = pltpu.async_copy(src_hbm.at[pl.ds(base_n, EC)],
                               src_v.at[pn], sem_i)
        cpw = pltpu.async_copy(w_hbm.at[pl.ds(c * E + base_n, EC)],
                               w_v.at[pn], sem_i)

        @pl.when(ch >= 2)
        def _drain():
            pltpu.make_async_copy(
                z128_hbm.at[pl.ds(0, EC)], x_v.at[p], sem_s).wait()

        @pl.loop(0, EC)
        def _expand(ei):
            x_v[p, ei, pl.ds(0, L)] = w_v[p, ei, :]

        pltpu.async_copy(x_v.at[p], acc_sh.at[src_v.at[p]], sem_s, add=True)
        cpi.wait()
        cpw.wait()

    for p2 in range(2):
        pltpu.make_async_copy(
            z128_hbm.at[pl.ds(0, EC)], x_v.at[p2], sem_s).wait()

    plsc.subcore_barrier()

    pltpu.sync_copy(acc_sh.at[pl.ds(row0, RPT)],
                    ssum_hbm.at[pl.ds(c * N + row0, RPT)])

    @pl.when(s < 2)
    def _dump_rem():
        r = NS * RPT + s * 8
        pltpu.sync_copy(acc_sh.at[pl.ds(r, 8)],
                        ssum_hbm.at[pl.ds(c * N + r, 8)])


def _sc_ssum(w2, src):
    mesh = plsc.VectorSubcoreMesh(core_axis_name="c", subcore_axis_name="s")
    fn = pl.kernel(
        _sc_ssum_body,
        out_type=[jax.ShapeDtypeStruct((NC * N, HALF), _F32)],
        mesh=mesh,
        scratch_types=[
            pltpu.VMEM((2, EC), _I32),
            pltpu.VMEM((2, EC, L), _F32),
            pltpu.VMEM((2, EC, HALF), _F32),
            pltpu.VMEM_SHARED((N, HALF), _F32),
            pltpu.SemaphoreType.DMA,
            pltpu.SemaphoreType.DMA,
        ],
    )
    z128 = jnp.zeros((N, HALF), _F32)
    return fn(w2, src, z128)[0]


def _proj_body(x_ref, w_ref, o_ref, nout):
    y = jnp.dot(x_ref[...], w_ref[...], preferred_element_type=_F32)
    for k in range(nout):
        o_ref[k, :, :] = y[:, k * HALF:(k + 1) * HALF]


def _proj(x, w, bn):
    rows, _ = x.shape
    nout = w.shape[1] // HALF
    grid = rows // bn
    return pl.pallas_call(
        functools.partial(_proj_body, nout=nout),
        grid=(grid,),
        in_specs=[
            pl.BlockSpec((bn, D), lambda i: (i, 0)),
            pl.BlockSpec((D, nout * HALF), lambda i: (0, 0)),
        ],
        out_specs=pl.BlockSpec((nout, bn, HALF), lambda i: (0, i, 0)),
        out_shape=jax.ShapeDtypeStruct((nout, rows, HALF), _F32),
    )(x, w)


def _stats_update(ref_sum, ref_sq, y, first):
    @pl.when(first)
    def _init():
        ref_sum[...] = jnp.zeros_like(ref_sum)
        ref_sq[...] = jnp.zeros_like(ref_sq)

    cs = jnp.sum(y, axis=0, keepdims=True)
    cq = jnp.sum(y * y, axis=0, keepdims=True)
    ref_sum[...] = ref_sum[...] + jnp.broadcast_to(cs, ref_sum.shape)
    ref_sq[...] = ref_sq[...] + jnp.broadcast_to(cq, ref_sq.shape)


def _post_a_h_body(a0, a1, s0, s1, xin, w_ref, b_ref,
                   y_ref, sum_ref, sq_ref):
    col = lax.broadcasted_iota(_I32, (L, HALF), 1) // DH
    row = lax.broadcasted_iota(_I32, (L, HALF), 0)
    bmat = (col == row).astype(_F32)
    s0v = s0[:, :L]
    s1v = s1[:, :L]
    r0 = jnp.where(s0v > 0, 1.0 / jnp.where(s0v > 0, s0v, 1.0), 0.0)
    r1 = jnp.where(s1v > 0, 1.0 / jnp.where(s1v > 0, s1v, 1.0), 0.0)
    att0 = a0[...] * jnp.dot(r0, bmat, preferred_element_type=_F32)
    att1 = a1[...] * jnp.dot(r1, bmat, preferred_element_type=_F32)
    att = jnp.concatenate([att0, att1], axis=1)
    y = jnp.dot(att, w_ref[...], preferred_element_type=_F32)
    y = y + b_ref[...] + xin[...]
    y_ref[...] = y
    _stats_update(sum_ref, sq_ref, y, pl.program_id(0) == 0)


def _post_a_e_body(a0, a1, xin, w_ref, b_ref, y_ref, sum_ref, sq_ref):
    att = jnp.concatenate([a0[...], a1[...]], axis=1)
    y = jnp.dot(att, w_ref[...], preferred_element_type=_F32)
    y = y + b_ref[...] + xin[...]
    y_ref[...] = y
    _stats_update(sum_ref, sq_ref, y, pl.program_id(0) == 0)


def _bn(y, sum_ref, sq_ref, g_ref, b_ref, rows):
    mean = sum_ref[0:1, :] * (1.0 / rows)
    var = sq_ref[0:1, :] * (1.0 / rows) - mean * mean
    inv = lax.rsqrt(var + 1e-5)
    return (y - mean) * inv * g_ref[...] + b_ref[...]


def _post_b_body(y_in, sum_ref, sq_ref, g_ref, b_ref, w1_ref, b1_ref,
                 w2_ref, b2_ref, f_ref, sum2_ref, sq2_ref, rows):
    z = _bn(y_in[...], sum_ref, sq_ref, g_ref, b_ref, rows)
    bf = jnp.bfloat16
    u = jnp.dot(z.astype(bf), w1_ref[...].astype(bf),
                preferred_element_type=_F32) + b1_ref[...]
    u = jnp.maximum(u, 0.0)
    f = jnp.dot(u.astype(bf), w2_ref[...].astype(bf),
                preferred_element_type=_F32) + b2_ref[...] + z
    f_ref[...] = f
    _stats_update(sum2_ref, sq2_ref, f, pl.program_id(0) == 0)


def _post_c_body(f_in, sum_ref, sq_ref, g_ref, b_ref, o_ref, rows):
    o_ref[...] = _bn(f_in[...], sum_ref, sq_ref, g_ref, b_ref, rows)


def _full_spec(shape):
    return pl.BlockSpec(shape, lambda i: tuple(0 for _ in shape))


def _post_a_h(hatt2, ssum2, h_in, wo, bo, bn):
    grid = N // bn
    blk = lambda r0: pl.BlockSpec((bn, HALF), lambda i, r0=r0: (r0 + i, 0))
    blks = blk
    return pl.pallas_call(
        _post_a_h_body,
        grid=(grid,),
        in_specs=[
            blk(0), blk(N // bn), blks(0), blks(N // bn),
            pl.BlockSpec((bn, D), lambda i: (i, 0)),
            _full_spec((D, D)), _full_spec((1, D)),
        ],
        out_specs=[
            pl.BlockSpec((bn, D), lambda i: (i, 0)),
            _full_spec((8, D)), _full_spec((8, D)),
        ],
        out_shape=[
            jax.ShapeDtypeStruct((N, D), _F32),
            jax.ShapeDtypeStruct((8, D), _F32),
            jax.ShapeDtypeStruct((8, D), _F32),
        ],
    )(hatt2, hatt2, ssum2, ssum2, h_in, wo, bo)


def _post_a_e(eout2, e_in, wo, bo, bn):
    grid = E // bn
    blk = lambda r0: pl.BlockSpec((bn, HALF), lambda i, r0=r0: (r0 + i, 0))
    return pl.pallas_call(
        _post_a_e_body,
        grid=(grid,),
        in_specs=[
            blk(0), blk(E // bn),
            pl.BlockSpec((bn, D), lambda i: (i, 0)),
            _full_spec((D, D)), _full_spec((1, D)),
        ],
        out_specs=[
            pl.BlockSpec((bn, D), lambda i: (i, 0)),
            _full_spec((8, D)), _full_spec((8, D)),
        ],
        out_shape=[
            jax.ShapeDtypeStruct((E, D), _F32),
            jax.ShapeDtypeStruct((8, D), _F32),
            jax.ShapeDtypeStruct((8, D), _F32),
        ],
    )(eout2, eout2, e_in, wo, bo)


def _post_b(y, sums, sqs, g, b, w1, b1, w2, b2, bn):
    rows = y.shape[0]
    grid = rows // bn
    return pl.pallas_call(
        functools.partial(_post_b_body, rows=float(rows)),
        grid=(grid,),
        in_specs=[
            pl.BlockSpec((bn, D), lambda i: (i, 0)),
            _full_spec((8, D)), _full_spec((8, D)),
            _full_spec((1, D)), _full_spec((1, D)),
            _full_spec((D, 2 * D)), _full_spec((1, 2 * D)),
            _full_spec((2 * D, D)), _full_spec((1, D)),
        ],
        out_specs=[
            pl.BlockSpec((bn, D), lambda i: (i, 0)),
            _full_spec((8, D)), _full_spec((8, D)),
        ],
        out_shape=[
            jax.ShapeDtypeStruct((rows, D), _F32),
            jax.ShapeDtypeStruct((8, D), _F32),
            jax.ShapeDtypeStruct((8, D), _F32),
        ],
    )(y, sums, sqs, g, b, w1, b1, w2, b2)


def _post_c(f, sums, sqs, g, b, bn):
    rows = f.shape[0]
    grid = rows // bn
    return pl.pallas_call(
        functools.partial(_post_c_body, rows=float(rows)),
        grid=(grid,),
        in_specs=[
            pl.BlockSpec((bn, D), lambda i: (i, 0)),
            _full_spec((8, D)), _full_spec((8, D)),
            _full_spec((1, D)), _full_spec((1, D)),
        ],
        out_specs=pl.BlockSpec((bn, D), lambda i: (i, 0)),
        out_shape=jax.ShapeDtypeStruct((rows, D), _F32),
    )(f, sums, sqs, g, b)


def kernel(h, e, edge_index, WQ, WK, WV, WE, WOh, bOh, WOe, bOe,
           bn1h_g, bn1h_b, bn1e_g, bn1e_b, W1h, b1h, W2h, b2h,
           W1e, b1e, W2e, b2e, bn2h_g, bn2h_b, bn2e_g, bn2e_b):
    src = edge_index[0].astype(_I32)
    dst = edge_index[1].astype(_I32)
    scale = 1.0 / math.sqrt(DH)

    wq = WQ.T
    wk = WK.T * scale
    wv = WV.T
    wkqv = jnp.concatenate([wk, wq, wv], axis=1)

    kqv = _proj(h, wkqv, 1000)
    k2 = kqv[0:2].reshape(NC * N, HALF)
    q2 = kqv[2:4].reshape(NC * N, HALF)
    v2 = kqv[4:6].reshape(NC * N, HALF)
    ee2 = _proj(e, WE.T, 2000).reshape(NC * E * HALF)

    eout2, hatt2, w2 = _sc_edge(k2, q2, v2, ee2, src, dst)
    eout2 = eout2.reshape(NC * E, HALF)
    ssum2 = _sc_ssum(w2, src)

    y_h, s1h, q1h = _post_a_h(hatt2, ssum2, h, WOh.T,
                              bOh.reshape(1, D), 1000)
    f_h, s2h, q2h = _post_b(y_h, s1h, q1h, bn1h_g.reshape(1, D),
                            bn1h_b.reshape(1, D), W1h.T, b1h.reshape(1, 2 * D),
                            W2h.T, b2h.reshape(1, D), 1000)
    h_out = _post_c(f_h, s2h, q2h, bn2h_g.reshape(1, D),
                    bn2h_b.reshape(1, D), 1000)

    y_e, s1e, q1e = _post_a_e(eout2, e, WOe.T, bOe.reshape(1, D), 2000)
    f_e, s2e, q2e = _post_b(y_e, s1e, q1e, bn1e_g.reshape(1, D),
                            bn1e_b.reshape(1, D), W1e.T, b1e.reshape(1, 2 * D),
                            W2e.T, b2e.reshape(1, D), 2000)
    e_out = _post_c(f_e, s2e, q2e, bn2e_g.reshape(1, D),
                    bn2e_b.reshape(1, D), 2000)

    return (h_out, e_out)


# speedup vs baseline: 4.2140x; 1.0480x over previous
import functools
import math

import jax
import jax.numpy as jnp
from jax import lax
from jax.experimental import pallas as pl
from jax.experimental.pallas import tpu as pltpu
from jax.experimental.pallas import tpu_sc as plsc

N = 10000
E = 160000
D = 256
H = 8
DH = 32
HALF = D // 2

NC = 2
NS = 16
L = 16

ET = E // NS
EC = 80
NCHUNK = ET // EC
RPT = 624
NZ = 208
NREM = N - NS * RPT

_F32 = jnp.float32
_I32 = jnp.int32


def _sc_edge_body(k_hbm, q_hbm, v_hbm, ee_hbm, src_hbm, dst_hbm,
                  z128_hbm,
                  eout_hbm, hatt_hbm, w_hbm,
                  srcd_v, adjs_v, kq_v, q_v, t_v, w_v,
                  acc_sh, sem_i, sem_g, sem_o, sem_s):
    s = lax.axis_index("s")
    c = lax.axis_index("c")

    row0 = s * RPT
    pltpu.sync_copy(z128_hbm.at[pl.ds(row0, RPT)], acc_sh.at[pl.ds(row0, RPT)])

    @pl.when(s < 2)
    def _zero_rem():
        r = NS * RPT + s * 8
        pltpu.sync_copy(z128_hbm.at[pl.ds(r, 8)], acc_sh.at[pl.ds(r, 8)])

    plsc.subcore_barrier()

    lanes = lax.iota(_I32, L)

    pltpu.sync_copy(src_hbm.at[pl.ds(s * ET, EC)], srcd_v.at[0, 0])
    pltpu.sync_copy(dst_hbm.at[pl.ds(s * ET, EC)], srcd_v.at[0, 1])

    @pl.loop(0, NCHUNK)
    def _chunk(ch):
        p = lax.rem(ch, 3)
        pn = lax.rem(ch + 1, 3)
        base = s * ET + ch * EC
        base_n = s * ET + jnp.minimum(ch + 1, NCHUNK - 1) * EC

        cpi1 = pltpu.async_copy(src_hbm.at[pl.ds(base_n, EC)],
                                srcd_v.at[pn, 0], sem_i)
        cpi2 = pltpu.async_copy(dst_hbm.at[pl.ds(base_n, EC)],
                                srcd_v.at[pn, 1], sem_i)

        off = c * N
        for g in range(EC // L):
            sl = pl.ds(g * L, L)
            adjs_v[sl] = srcd_v[p, 0, sl] + off
            srcd_v[p, 1, sl] = srcd_v[p, 1, sl] + off

        @pl.when(ch >= 1)
        def _drain_sct():
            pltpu.make_async_copy(
                z128_hbm.at[pl.ds(0, EC)], kq_v, sem_s).wait()

        cp_k = pltpu.async_copy(k_hbm.at[adjs_v], kq_v, sem_g)
        cp_q = pltpu.async_copy(q_hbm.at[srcd_v.at[p, 1]], q_v, sem_g)
        cp_e = pltpu.async_copy(
            ee_hbm.at[pl.ds((c * E + base) * HALF, EC * HALF)], t_v, sem_g)
        cp_k.wait()
        cp_q.wait()
        cp_e.wait()

        @pl.loop(0, EC)
        def _tmul(ei):
            tbase = ei * HALF
            for j in range(HALF // L):
                tsl = pl.ds(tbase + j * L, L)
                t_v[tsl] = kq_v[ei, pl.ds(j * L, L)] * \
                    q_v[ei, pl.ds(j * L, L)] * t_v[tsl]

        cp_v = pltpu.async_copy(v_hbm.at[srcd_v.at[p, 1]], kq_v, sem_g)
        cp_out = pltpu.async_copy(
            t_v, eout_hbm.at[pl.ds((c * E + base) * HALF, EC * HALF)], sem_o)

        perm1 = lanes ^ 1
        perm2 = lanes ^ 2
        perm4 = lanes ^ 4
        perm8 = lanes ^ 8
        lm4 = lanes & 3

        @pl.loop(0, EC)
        def _score(ei):
            tbase = ei * HALF
            a = []
            for hh in range(H // 2):
                x = t_v[pl.ds(tbase + 2 * hh * L, L)] + \
                    t_v[pl.ds(tbase + (2 * hh + 1) * L, L)]
                x = x + x.at[perm1].get(mode="promise_in_bounds")
                x = x + x.at[perm2].get(mode="promise_in_bounds")
                a.append(x)
            comb = jnp.where(lm4 == 0, a[0],
                             jnp.where(lm4 == 1, a[1],
                                       jnp.where(lm4 == 2, a[2], a[3])))
            comb = comb + comb.at[perm4].get(mode="promise_in_bounds")
            comb = comb + comb.at[perm8].get(mode="promise_in_bounds")
            w_v[ei, :] = jnp.exp(comb)

        cp_w = pltpu.async_copy(w_v, w_hbm.at[pl.ds(c * E + base, EC)], sem_o)
        cp_v.wait()

        @pl.loop(0, EC)
        def _wv(ei):
            wrow = w_v[ei, :]
            for hh in range(H // 2):
                bc = jnp.full((L,), wrow[hh], _F32)
                for j2 in range(DH // L):
                    sl = pl.ds(hh * DH + j2 * L, L)
                    kq_v[ei, sl] = kq_v[ei, sl] * bc

        pltpu.async_copy(kq_v, acc_sh.at[srcd_v.at[p, 0]], sem_s, add=True)
        cp_out.wait()
        cp_w.wait()
        cpi1.wait()
        cpi2.wait()

    pltpu.make_async_copy(z128_hbm.at[pl.ds(0, EC)], kq_v, sem_s).wait()

    plsc.subcore_barrier()

    pltpu.sync_copy(acc_sh.at[pl.ds(row0, RPT)],
                    hatt_hbm.at[pl.ds(c * N + row0, RPT)])

    @pl.when(s < 2)
    def _dump_rem():
        r = NS * RPT + s * 8
        pltpu.sync_copy(acc_sh.at[pl.ds(r, 8)],
                        hatt_hbm.at[pl.ds(c * N + r, 8)])


def _sc_edge(k2, q2, v2, ee2, src, dst):
    mesh = plsc.VectorSubcoreMesh(core_axis_name="c", subcore_axis_name="s")
    fn = pl.kernel(
        _sc_edge_body,
        out_type=[
            jax.ShapeDtypeStruct((NC * E * HALF,), _F32),
            jax.ShapeDtypeStruct((NC * N, HALF), _F32),
            jax.ShapeDtypeStruct((NC * E, L), _F32),
        ],
        mesh=mesh,
        scratch_types=[
            pltpu.VMEM((3, 2, EC), _I32),
            pltpu.VMEM((EC,), _I32),
            pltpu.VMEM((EC, HALF), _F32),
            pltpu.VMEM((EC, HALF), _F32),
            pltpu.VMEM((EC * HALF,), _F32),
            pltpu.VMEM((EC, L), _F32),
            pltpu.VMEM_SHARED((N, HALF), _F32),
            pltpu.SemaphoreType.DMA,
            pltpu.SemaphoreType.DMA,
            pltpu.SemaphoreType.DMA,
            pltpu.SemaphoreType.DMA,
        ],
    )
    z128 = jnp.zeros((N, HALF), _F32)
    return fn(k2, q2, v2, ee2, src, dst, z128)


def _sc_ssum_body(w_hbm, src_hbm, z128_hbm, ssum_hbm,
                  src_v, w_v, x_v, acc_sh, sem_i, sem_s):
    s = lax.axis_index("s")
    c = lax.axis_index("c")

    row0 = s * RPT
    pltpu.sync_copy(z128_hbm.at[pl.ds(row0, RPT)], acc_sh.at[pl.ds(row0, RPT)])

    @pl.when(s < 2)
    def _zero_rem():
        r = NS * RPT + s * 8
        pltpu.sync_copy(z128_hbm.at[pl.ds(r, 8)], acc_sh.at[pl.ds(r, 8)])

    @pl.loop(0, EC)
    def _zero_x(ei):
        for p2 in range(2):
            for j in range(HALF // L):
                x_v[p2, ei, pl.ds(j * L, L)] = jnp.zeros((L,), _F32)

    plsc.subcore_barrier()

    pltpu.sync_copy(src_hbm.at[pl.ds(s * ET, EC)], src_v.at[0])
    pltpu.sync_copy(w_hbm.at[pl.ds(c * E + s * ET, EC)], w_v.at[0])

    @pl.loop(0, NCHUNK)
    def _chunk(ch):
        p = lax.rem(ch, 2)
        pn = 1 - p
        base_n = s * ET + jnp.minimum(ch + 1, NCHUNK - 1) * EC
        cpi = pltpu.async_copy(src_hbm.at[pl.ds(base_n, EC)],
                               src_v.at[pn], sem_i)
        cpw = pltpu.async_copy(w_hbm.at[pl.ds(c * E + base_n, EC)],
                               w_v.at[pn], sem_i)

        @pl.when(ch >= 2)
        def _drain():
            pltpu.make_async_copy(
                z128_hbm.at[pl.ds(0, EC)], x_v.at[p], sem_s).wait()

        @pl.loop(0, EC)
        def _expand(ei):
            x_v[p, ei, pl.ds(0, L)] = w_v[p, ei, :]

        pltpu.async_copy(x_v.at[p], acc_sh.at[src_v.at[p]], sem_s, add=True)
        cpi.wait()
        cpw.wait()

    for p2 in range(2):
        pltpu.make_async_copy(
            z128_hbm.at[pl.ds(0, EC)], x_v.at[p2], sem_s).wait()

    plsc.subcore_barrier()

    pltpu.sync_copy(acc_sh.at[pl.ds(row0, RPT)],
                    ssum_hbm.at[pl.ds(c * N + row0, RPT)])

    @pl.when(s < 2)
    def _dump_rem():
        r = NS * RPT + s * 8
        pltpu.sync_copy(acc_sh.at[pl.ds(r, 8)],
                        ssum_hbm.at[pl.ds(c * N + r, 8)])


def _sc_ssum(w2, src):
    mesh = plsc.VectorSubcoreMesh(core_axis_name="c", subcore_axis_name="s")
    fn = pl.kernel(
        _sc_ssum_body,
        out_type=[jax.ShapeDtypeStruct((NC * N, HALF), _F32)],
        mesh=mesh,
        scratch_types=[
            pltpu.VMEM((2, EC), _I32),
            pltpu.VMEM((2, EC, L), _F32),
            pltpu.VMEM((2, EC, HALF), _F32),
            pltpu.VMEM_SHARED((N, HALF), _F32),
            pltpu.SemaphoreType.DMA,
            pltpu.SemaphoreType.DMA,
        ],
    )
    z128 = jnp.zeros((N, HALF), _F32)
    return fn(w2, src, z128)[0]


def _proj_body(x_ref, w_ref, o_ref, nout):
    y = jnp.dot(x_ref[...], w_ref[...], preferred_element_type=_F32)
    for k in range(nout):
        o_ref[k, :, :] = y[:, k * HALF:(k + 1) * HALF]


def _proj(x, w, bn):
    rows, _ = x.shape
    nout = w.shape[1] // HALF
    grid = rows // bn
    return pl.pallas_call(
        functools.partial(_proj_body, nout=nout),
        grid=(grid,),
        in_specs=[
            pl.BlockSpec((bn, D), lambda i: (i, 0)),
            pl.BlockSpec((D, nout * HALF), lambda i: (0, 0)),
        ],
        out_specs=pl.BlockSpec((nout, bn, HALF), lambda i: (0, i, 0)),
        out_shape=jax.ShapeDtypeStruct((nout, rows, HALF), _F32),
    )(x, w)


def _stats_update(ref_sum, ref_sq, y, first):
    @pl.when(first)
    def _init():
        ref_sum[...] = jnp.zeros_like(ref_sum)
        ref_sq[...] = jnp.zeros_like(ref_sq)

    cs = jnp.sum(y, axis=0, keepdims=True)
    cq = jnp.sum(y * y, axis=0, keepdims=True)
    ref_sum[...] = ref_sum[...] + jnp.broadcast_to(cs, ref_sum.shape)
    ref_sq[...] = ref_sq[...] + jnp.broadcast_to(cq, ref_sq.shape)


def _post_a_h_body(a0, a1, s0, s1, xin, w_ref, b_ref,
                   y_ref, sum_ref, sq_ref):
    col = lax.broadcasted_iota(_I32, (L, HALF), 1) // DH
    row = lax.broadcasted_iota(_I32, (L, HALF), 0)
    bmat = (col == row).astype(_F32)
    s0v = s0[:, :L]
    s1v = s1[:, :L]
    r0 = jnp.where(s0v > 0, 1.0 / jnp.where(s0v > 0, s0v, 1.0), 0.0)
    r1 = jnp.where(s1v > 0, 1.0 / jnp.where(s1v > 0, s1v, 1.0), 0.0)
    att0 = a0[...] * jnp.dot(r0, bmat, preferred_element_type=_F32)
    att1 = a1[...] * jnp.dot(r1, bmat, preferred_element_type=_F32)
    att = jnp.concatenate([att0, att1], axis=1)
    y = jnp.dot(att, w_ref[...], preferred_element_type=_F32)
    y = y + b_ref[...] + xin[...]
    y_ref[...] = y
    _stats_update(sum_ref, sq_ref, y, pl.program_id(0) == 0)


def _post_a_e_body(a0, a1, xin, w_ref, b_ref, y_ref, sum_ref, sq_ref):
    att = jnp.concatenate([a0[...], a1[...]], axis=1)
    y = jnp.dot(att, w_ref[...], preferred_element_type=_F32)
    y = y + b_ref[...] + xin[...]
    y_ref[...] = y.astype(y_ref.dtype)
    _stats_update(sum_ref, sq_ref, y, pl.program_id(0) == 0)


def _bn(y, sum_ref, sq_ref, g_ref, b_ref, rows):
    mean = sum_ref[0:1, :] * (1.0 / rows)
    var = sq_ref[0:1, :] * (1.0 / rows) - mean * mean
    inv = lax.rsqrt(var + 1e-5)
    return (y - mean) * inv * g_ref[...] + b_ref[...]


def _post_b_body(y_in, sum_ref, sq_ref, g_ref, b_ref, w1_ref, b1_ref,
                 w2_ref, b2_ref, f_ref, sum2_ref, sq2_ref, rows):
    z = _bn(y_in[...].astype(_F32), sum_ref, sq_ref, g_ref, b_ref, rows)
    u = jnp.dot(z, w1_ref[...], preferred_element_type=_F32) + b1_ref[...]
    u = jnp.maximum(u, 0.0)
    f = jnp.dot(u, w2_ref[...], preferred_element_type=_F32) + b2_ref[...] + z
    f_ref[...] = f.astype(f_ref.dtype)
    _stats_update(sum2_ref, sq2_ref, f, pl.program_id(0) == 0)


def _post_c_body(f_in, sum_ref, sq_ref, g_ref, b_ref, o_ref, rows):
    o_ref[...] = _bn(f_in[...].astype(_F32), sum_ref, sq_ref, g_ref, b_ref,
                     rows)


def _full_spec(shape):
    return pl.BlockSpec(shape, lambda i: tuple(0 for _ in shape))


def _post_a_h(hatt2, ssum2, h_in, wo, bo, bn):
    grid = N // bn
    blk = lambda r0: pl.BlockSpec((bn, HALF), lambda i, r0=r0: (r0 + i, 0))
    blks = blk
    return pl.pallas_call(
        _post_a_h_body,
        grid=(grid,),
        in_specs=[
            blk(0), blk(N // bn), blks(0), blks(N // bn),
            pl.BlockSpec((bn, D), lambda i: (i, 0)),
            _full_spec((D, D)), _full_spec((1, D)),
        ],
        out_specs=[
            pl.BlockSpec((bn, D), lambda i: (i, 0)),
            _full_spec((8, D)), _full_spec((8, D)),
        ],
        out_shape=[
            jax.ShapeDtypeStruct((N, D), _F32),
            jax.ShapeDtypeStruct((8, D), _F32),
            jax.ShapeDtypeStruct((8, D), _F32),
        ],
    )(hatt2, hatt2, ssum2, ssum2, h_in, wo, bo)


def _post_a_e(eout2, e_in, wo, bo, bn, ydt=_F32):
    grid = E // bn
    blk = lambda r0: pl.BlockSpec((bn, HALF), lambda i, r0=r0: (r0 + i, 0))
    return pl.pallas_call(
        _post_a_e_body,
        grid=(grid,),
        in_specs=[
            blk(0), blk(E // bn),
            pl.BlockSpec((bn, D), lambda i: (i, 0)),
            _full_spec((D, D)), _full_spec((1, D)),
        ],
        out_specs=[
            pl.BlockSpec((bn, D), lambda i: (i, 0)),
            _full_spec((8, D)), _full_spec((8, D)),
        ],
        out_shape=[
            jax.ShapeDtypeStruct((E, D), ydt),
            jax.ShapeDtypeStruct((8, D), _F32),
            jax.ShapeDtypeStruct((8, D), _F32),
        ],
    )(eout2, eout2, e_in, wo, bo)


def _post_b(y, sums, sqs, g, b, w1, b1, w2, b2, bn, fdt=_F32):
    rows = y.shape[0]
    grid = rows // bn
    return pl.pallas_call(
        functools.partial(_post_b_body, rows=float(rows)),
        grid=(grid,),
        in_specs=[
            pl.BlockSpec((bn, D), lambda i: (i, 0)),
            _full_spec((8, D)), _full_spec((8, D)),
            _full_spec((1, D)), _full_spec((1, D)),
            _full_spec((D, 2 * D)), _full_spec((1, 2 * D)),
            _full_spec((2 * D, D)), _full_spec((1, D)),
        ],
        out_specs=[
            pl.BlockSpec((bn, D), lambda i: (i, 0)),
            _full_spec((8, D)), _full_spec((8, D)),
        ],
        out_shape=[
            jax.ShapeDtypeStruct((rows, D), fdt),
            jax.ShapeDtypeStruct((8, D), _F32),
            jax.ShapeDtypeStruct((8, D), _F32),
        ],
    )(y, sums, sqs, g, b, w1, b1, w2, b2)


def _post_c(f, sums, sqs, g, b, bn):
    rows = f.shape[0]
    grid = rows // bn
    return pl.pallas_call(
        functools.partial(_post_c_body, rows=float(rows)),
        grid=(grid,),
        in_specs=[
            pl.BlockSpec((bn, D), lambda i: (i, 0)),
            _full_spec((8, D)), _full_spec((8, D)),
            _full_spec((1, D)), _full_spec((1, D)),
        ],
        out_specs=pl.BlockSpec((bn, D), lambda i: (i, 0)),
        out_shape=jax.ShapeDtypeStruct((rows, D), _F32),
    )(f, sums, sqs, g, b)


def kernel(h, e, edge_index, WQ, WK, WV, WE, WOh, bOh, WOe, bOe,
           bn1h_g, bn1h_b, bn1e_g, bn1e_b, W1h, b1h, W2h, b2h,
           W1e, b1e, W2e, b2e, bn2h_g, bn2h_b, bn2e_g, bn2e_b):
    src = edge_index[0].astype(_I32)
    dst = edge_index[1].astype(_I32)
    scale = 1.0 / math.sqrt(DH)

    wq = WQ.T
    wk = WK.T * scale
    wv = WV.T
    wkqv = jnp.concatenate([wk, wq, wv], axis=1)

    kqv = _proj(h, wkqv, 1000)
    k2 = kqv[0:2].reshape(NC * N, HALF)
    q2 = kqv[2:4].reshape(NC * N, HALF)
    v2 = kqv[4:6].reshape(NC * N, HALF)
    ee2 = _proj(e, WE.T, 2000).reshape(NC * E * HALF)

    eout2, hatt2, w2 = _sc_edge(k2, q2, v2, ee2, src, dst)
    eout2 = eout2.reshape(NC * E, HALF)
    ssum2 = _sc_ssum(w2, src)

    y_h, s1h, q1h = _post_a_h(hatt2, ssum2, h, WOh.T,
                              bOh.reshape(1, D), 1000)
    f_h, s2h, q2h = _post_b(y_h, s1h, q1h, bn1h_g.reshape(1, D),
                            bn1h_b.reshape(1, D), W1h.T, b1h.reshape(1, 2 * D),
                            W2h.T, b2h.reshape(1, D), 1000)
    h_out = _post_c(f_h, s2h, q2h, bn2h_g.reshape(1, D),
                    bn2h_b.reshape(1, D), 1000)

    y_e, s1e, q1e = _post_a_e(eout2, e, WOe.T, bOe.reshape(1, D), 2000,
                              ydt=jnp.bfloat16)
    f_e, s2e, q2e = _post_b(y_e, s1e, q1e, bn1e_g.reshape(1, D),
                            bn1e_b.reshape(1, D), W1e.T, b1e.reshape(1, 2 * D),
                            W2e.T, b2e.reshape(1, D), 2000, fdt=jnp.bfloat16)
    e_out = _post_c(f_e, s2e, q2e, bn2e_g.reshape(1, D),
                    bn2e_b.reshape(1, D), 2000)

    return (h_out, e_out)


# speedup vs baseline: 4.3588x; 1.0344x over previous
import functools
import math

import jax
import jax.numpy as jnp
from jax import lax
from jax.experimental import pallas as pl
from jax.experimental.pallas import tpu as pltpu
from jax.experimental.pallas import tpu_sc as plsc

N = 10000
E = 160000
D = 256
H = 8
DH = 32
HALF = D // 2

NC = 2
NS = 16
L = 16

ET = E // NS
EC = 80
NCHUNK = ET // EC
RPT = 624
NZ = 208
NREM = N - NS * RPT

_F32 = jnp.float32
_I32 = jnp.int32


def _sc_edge_body(k_hbm, q_hbm, v_hbm, ee_hbm, src_hbm, dst_hbm,
                  z128_hbm,
                  eout_hbm, hatt_hbm, w_hbm,
                  srcd_v, adjs_v, kq_v, q_v, t_v, w_v,
                  acc_sh, sem_i, sem_g, sem_o, sem_s):
    s = lax.axis_index("s")
    c = lax.axis_index("c")

    row0 = s * RPT
    pltpu.sync_copy(z128_hbm.at[pl.ds(row0, RPT)], acc_sh.at[pl.ds(row0, RPT)])

    @pl.when(s < 2)
    def _zero_rem():
        r = NS * RPT + s * 8
        pltpu.sync_copy(z128_hbm.at[pl.ds(r, 8)], acc_sh.at[pl.ds(r, 8)])

    plsc.subcore_barrier()

    lanes = lax.iota(_I32, L)

    pltpu.sync_copy(src_hbm.at[pl.ds(s * ET, EC)], srcd_v.at[0, 0])
    pltpu.sync_copy(dst_hbm.at[pl.ds(s * ET, EC)], srcd_v.at[0, 1])

    @pl.loop(0, NCHUNK)
    def _chunk(ch):
        p = lax.rem(ch, 3)
        pn = lax.rem(ch + 1, 3)
        base = s * ET + ch * EC
        base_n = s * ET + jnp.minimum(ch + 1, NCHUNK - 1) * EC

        cpi1 = pltpu.async_copy(src_hbm.at[pl.ds(base_n, EC)],
                                srcd_v.at[pn, 0], sem_i)
        cpi2 = pltpu.async_copy(dst_hbm.at[pl.ds(base_n, EC)],
                                srcd_v.at[pn, 1], sem_i)

        off = c * N
        for g in range(EC // L):
            sl = pl.ds(g * L, L)
            adjs_v[sl] = srcd_v[p, 0, sl] + off
            srcd_v[p, 1, sl] = srcd_v[p, 1, sl] + off

        cp_q = pltpu.async_copy(q_hbm.at[srcd_v.at[p, 1]], q_v, sem_g)
        cp_e = pltpu.async_copy(
            ee_hbm.at[pl.ds((c * E + base) * HALF, EC * HALF)], t_v, sem_g)

        @pl.when(ch >= 1)
        def _drain_sct():
            pltpu.make_async_copy(
                z128_hbm.at[pl.ds(0, EC)], kq_v, sem_s).wait()

        cp_k = pltpu.async_copy(k_hbm.at[adjs_v], kq_v, sem_g)
        cp_k.wait()
        cp_q.wait()
        cp_e.wait()

        @pl.loop(0, EC)
        def _tmul(ei):
            tbase = ei * HALF
            for j in range(HALF // L):
                tsl = pl.ds(tbase + j * L, L)
                t_v[tsl] = kq_v[ei, pl.ds(j * L, L)] * \
                    q_v[ei, pl.ds(j * L, L)] * t_v[tsl]

        cp_v = pltpu.async_copy(v_hbm.at[srcd_v.at[p, 1]], kq_v, sem_g)
        cp_out = pltpu.async_copy(
            t_v, eout_hbm.at[pl.ds((c * E + base) * HALF, EC * HALF)], sem_o)

        perm1 = lanes ^ 1
        perm2 = lanes ^ 2
        perm4 = lanes ^ 4
        perm8 = lanes ^ 8
        lm4 = lanes & 3

        @pl.loop(0, EC)
        def _score(ei):
            tbase = ei * HALF
            a = []
            for hh in range(H // 2):
                x = t_v[pl.ds(tbase + 2 * hh * L, L)] + \
                    t_v[pl.ds(tbase + (2 * hh + 1) * L, L)]
                x = x + x.at[perm1].get(mode="promise_in_bounds")
                x = x + x.at[perm2].get(mode="promise_in_bounds")
                a.append(x)
            comb = jnp.where(lm4 == 0, a[0],
                             jnp.where(lm4 == 1, a[1],
                                       jnp.where(lm4 == 2, a[2], a[3])))
            comb = comb + comb.at[perm4].get(mode="promise_in_bounds")
            comb = comb + comb.at[perm8].get(mode="promise_in_bounds")
            w_v[ei, :] = jnp.exp(comb)

        cp_w = pltpu.async_copy(w_v, w_hbm.at[pl.ds(c * E + base, EC)], sem_o)
        cp_v.wait()

        @pl.loop(0, EC)
        def _wv(ei):
            wrow = w_v[ei, :]
            for hh in range(H // 2):
                bc = jnp.full((L,), wrow[hh], _F32)
                for j2 in range(DH // L):
                    sl = pl.ds(hh * DH + j2 * L, L)
                    kq_v[ei, sl] = kq_v[ei, sl] * bc

        pltpu.async_copy(kq_v, acc_sh.at[srcd_v.at[p, 0]], sem_s, add=True)
        cp_out.wait()
        cp_w.wait()
        cpi1.wait()
        cpi2.wait()

    pltpu.make_async_copy(z128_hbm.at[pl.ds(0, EC)], kq_v, sem_s).wait()

    plsc.subcore_barrier()

    pltpu.sync_copy(acc_sh.at[pl.ds(row0, RPT)],
                    hatt_hbm.at[pl.ds(c * N + row0, RPT)])

    @pl.when(s < 2)
    def _dump_rem():
        r = NS * RPT + s * 8
        pltpu.sync_copy(acc_sh.at[pl.ds(r, 8)],
                        hatt_hbm.at[pl.ds(c * N + r, 8)])


def _sc_edge(k2, q2, v2, ee2, src, dst):
    mesh = plsc.VectorSubcoreMesh(core_axis_name="c", subcore_axis_name="s")
    fn = pl.kernel(
        _sc_edge_body,
        out_type=[
            jax.ShapeDtypeStruct((NC * E * HALF,), _F32),
            jax.ShapeDtypeStruct((NC * N, HALF), _F32),
            jax.ShapeDtypeStruct((NC * E, L), _F32),
        ],
        mesh=mesh,
        scratch_types=[
            pltpu.VMEM((3, 2, EC), _I32),
            pltpu.VMEM((EC,), _I32),
            pltpu.VMEM((EC, HALF), _F32),
            pltpu.VMEM((EC, HALF), _F32),
            pltpu.VMEM((EC * HALF,), _F32),
            pltpu.VMEM((EC, L), _F32),
            pltpu.VMEM_SHARED((N, HALF), _F32),
            pltpu.SemaphoreType.DMA,
            pltpu.SemaphoreType.DMA,
            pltpu.SemaphoreType.DMA,
            pltpu.SemaphoreType.DMA,
        ],
    )
    z128 = jnp.zeros((N, HALF), _F32)
    return fn(k2, q2, v2, ee2, src, dst, z128)


def _sc_ssum_body(w_hbm, src_hbm, z128_hbm, ssum_hbm,
                  src_v, w_v, x_v, acc_sh, sem_i, sem_s):
    s = lax.axis_index("s")
    c = lax.axis_index("c")

    row0 = s * RPT
    pltpu.sync_copy(z128_hbm.at[pl.ds(row0, RPT)], acc_sh.at[pl.ds(row0, RPT)])

    @pl.when(s < 2)
    def _zero_rem():
        r = NS * RPT + s * 8
        pltpu.sync_copy(z128_hbm.at[pl.ds(r, 8)], acc_sh.at[pl.ds(r, 8)])

    @pl.loop(0, EC)
    def _zero_x(ei):
        for p2 in range(2):
            for j in range(HALF // L):
                x_v[p2, ei, pl.ds(j * L, L)] = jnp.zeros((L,), _F32)

    plsc.subcore_barrier()

    pltpu.sync_copy(src_hbm.at[pl.ds(s * ET, EC)], src_v.at[0])
    pltpu.sync_copy(w_hbm.at[pl.ds(c * E + s * ET, EC)], w_v.at[0])

    @pl.loop(0, NCHUNK)
    def _chunk(ch):
        p = lax.rem(ch, 2)
        pn = 1 - p
        base_n = s * ET + jnp.minimum(ch + 1, NCHUNK - 1) * EC
        cpi = pltpu.async_copy(src_hbm.at[pl.ds(base_n, EC)],
                               src_v.at[pn], sem_i)
        cpw = pltpu.async_copy(w_hbm.at[pl.ds(c * E + base_n, EC)],
                               w_v.at[pn], sem_i)

        @pl.when(ch >= 2)
        def _drain():
            pltpu.make_async_copy(
                z128_hbm.at[pl.ds(0, EC)], x_v.at[p], sem_s).wait()

        @pl.loop(0, EC)
        def _expand(ei):
            x_v[p, ei, pl.ds(0, L)] = w_v[p, ei, :]

        pltpu.async_copy(x_v.at[p], acc_sh.at[src_v.at[p]], sem_s, add=True)
        cpi.wait()
        cpw.wait()

    for p2 in range(2):
        pltpu.make_async_copy(
            z128_hbm.at[pl.ds(0, EC)], x_v.at[p2], sem_s).wait()

    plsc.subcore_barrier()

    pltpu.sync_copy(acc_sh.at[pl.ds(row0, RPT)],
                    ssum_hbm.at[pl.ds(c * N + row0, RPT)])

    @pl.when(s < 2)
    def _dump_rem():
        r = NS * RPT + s * 8
        pltpu.sync_copy(acc_sh.at[pl.ds(r, 8)],
                        ssum_hbm.at[pl.ds(c * N + r, 8)])


def _sc_ssum(w2, src):
    mesh = plsc.VectorSubcoreMesh(core_axis_name="c", subcore_axis_name="s")
    fn = pl.kernel(
        _sc_ssum_body,
        out_type=[jax.ShapeDtypeStruct((NC * N, HALF), _F32)],
        mesh=mesh,
        scratch_types=[
            pltpu.VMEM((2, EC), _I32),
            pltpu.VMEM((2, EC, L), _F32),
            pltpu.VMEM((2, EC, HALF), _F32),
            pltpu.VMEM_SHARED((N, HALF), _F32),
            pltpu.SemaphoreType.DMA,
            pltpu.SemaphoreType.DMA,
        ],
    )
    z128 = jnp.zeros((N, HALF), _F32)
    return fn(w2, src, z128)[0]


def _proj_body(x_ref, w_ref, o_ref, nout):
    y = jnp.dot(x_ref[...], w_ref[...], preferred_element_type=_F32)
    for k in range(nout):
        o_ref[k, :, :] = y[:, k * HALF:(k + 1) * HALF]


def _proj(x, w, bn):
    rows, _ = x.shape
    nout = w.shape[1] // HALF
    grid = rows // bn
    return pl.pallas_call(
        functools.partial(_proj_body, nout=nout),
        grid=(grid,),
        in_specs=[
            pl.BlockSpec((bn, D), lambda i: (i, 0)),
            pl.BlockSpec((D, nout * HALF), lambda i: (0, 0)),
        ],
        out_specs=pl.BlockSpec((nout, bn, HALF), lambda i: (0, i, 0)),
        out_shape=jax.ShapeDtypeStruct((nout, rows, HALF), _F32),
    )(x, w)


def _stats_update(ref_sum, ref_sq, y, first):
    @pl.when(first)
    def _init():
        ref_sum[...] = jnp.zeros_like(ref_sum)
        ref_sq[...] = jnp.zeros_like(ref_sq)

    cs = jnp.sum(y, axis=0, keepdims=True)
    cq = jnp.sum(y * y, axis=0, keepdims=True)
    ref_sum[...] = ref_sum[...] + jnp.broadcast_to(cs, ref_sum.shape)
    ref_sq[...] = ref_sq[...] + jnp.broadcast_to(cq, ref_sq.shape)


def _post_a_h_body(a0, a1, s0, s1, xin, w_ref, b_ref,
                   y_ref, sum_ref, sq_ref):
    col = lax.broadcasted_iota(_I32, (L, HALF), 1) // DH
    row = lax.broadcasted_iota(_I32, (L, HALF), 0)
    bmat = (col == row).astype(_F32)
    s0v = s0[:, :L]
    s1v = s1[:, :L]
    r0 = jnp.where(s0v > 0, 1.0 / jnp.where(s0v > 0, s0v, 1.0), 0.0)
    r1 = jnp.where(s1v > 0, 1.0 / jnp.where(s1v > 0, s1v, 1.0), 0.0)
    att0 = a0[...] * jnp.dot(r0, bmat, preferred_element_type=_F32)
    att1 = a1[...] * jnp.dot(r1, bmat, preferred_element_type=_F32)
    att = jnp.concatenate([att0, att1], axis=1)
    y = jnp.dot(att, w_ref[...], preferred_element_type=_F32)
    y = y + b_ref[...] + xin[...]
    y_ref[...] = y
    _stats_update(sum_ref, sq_ref, y, pl.program_id(0) == 0)


def _post_a_e_body(a0, a1, xin, w_ref, b_ref, y_ref, sum_ref, sq_ref):
    att = jnp.concatenate([a0[...], a1[...]], axis=1)
    y = jnp.dot(att, w_ref[...], preferred_element_type=_F32)
    y = y + b_ref[...] + xin[...]
    y_ref[...] = y.astype(y_ref.dtype)
    _stats_update(sum_ref, sq_ref, y, pl.program_id(0) == 0)


def _bn(y, sum_ref, sq_ref, g_ref, b_ref, rows):
    mean = sum_ref[0:1, :] * (1.0 / rows)
    var = sq_ref[0:1, :] * (1.0 / rows) - mean * mean
    inv = lax.rsqrt(var + 1e-5)
    return (y - mean) * inv * g_ref[...] + b_ref[...]


def _post_b_body(y_in, sum_ref, sq_ref, g_ref, b_ref, w1_ref, b1_ref,
                 w2_ref, b2_ref, f_ref, sum2_ref, sq2_ref, rows):
    z = _bn(y_in[...].astype(_F32), sum_ref, sq_ref, g_ref, b_ref, rows)
    u = jnp.dot(z, w1_ref[...], preferred_element_type=_F32) + b1_ref[...]
    u = jnp.maximum(u, 0.0)
    f = jnp.dot(u, w2_ref[...], preferred_element_type=_F32) + b2_ref[...] + z
    f_ref[...] = f.astype(f_ref.dtype)
    _stats_update(sum2_ref, sq2_ref, f, pl.program_id(0) == 0)


def _post_c_body(f_in, sum_ref, sq_ref, g_ref, b_ref, o_ref, rows):
    o_ref[...] = _bn(f_in[...].astype(_F32), sum_ref, sq_ref, g_ref, b_ref,
                     rows)


def _full_spec(shape):
    return pl.BlockSpec(shape, lambda i: tuple(0 for _ in shape))


def _post_a_h(hatt2, ssum2, h_in, wo, bo, bn):
    grid = N // bn
    blk = lambda r0: pl.BlockSpec((bn, HALF), lambda i, r0=r0: (r0 + i, 0))
    blks = blk
    return pl.pallas_call(
        _post_a_h_body,
        grid=(grid,),
        in_specs=[
            blk(0), blk(N // bn), blks(0), blks(N // bn),
            pl.BlockSpec((bn, D), lambda i: (i, 0)),
            _full_spec((D, D)), _full_spec((1, D)),
        ],
        out_specs=[
            pl.BlockSpec((bn, D), lambda i: (i, 0)),
            _full_spec((8, D)), _full_spec((8, D)),
        ],
        out_shape=[
            jax.ShapeDtypeStruct((N, D), _F32),
            jax.ShapeDtypeStruct((8, D), _F32),
            jax.ShapeDtypeStruct((8, D), _F32),
        ],
    )(hatt2, hatt2, ssum2, ssum2, h_in, wo, bo)


def _post_a_e(eout2, e_in, wo, bo, bn, ydt=_F32):
    grid = E // bn
    blk = lambda r0: pl.BlockSpec((bn, HALF), lambda i, r0=r0: (r0 + i, 0))
    return pl.pallas_call(
        _post_a_e_body,
        grid=(grid,),
        in_specs=[
            blk(0), blk(E // bn),
            pl.BlockSpec((bn, D), lambda i: (i, 0)),
            _full_spec((D, D)), _full_spec((1, D)),
        ],
        out_specs=[
            pl.BlockSpec((bn, D), lambda i: (i, 0)),
            _full_spec((8, D)), _full_spec((8, D)),
        ],
        out_shape=[
            jax.ShapeDtypeStruct((E, D), ydt),
            jax.ShapeDtypeStruct((8, D), _F32),
            jax.ShapeDtypeStruct((8, D), _F32),
        ],
    )(eout2, eout2, e_in, wo, bo)


def _post_b(y, sums, sqs, g, b, w1, b1, w2, b2, bn, fdt=_F32):
    rows = y.shape[0]
    grid = rows // bn
    return pl.pallas_call(
        functools.partial(_post_b_body, rows=float(rows)),
        grid=(grid,),
        in_specs=[
            pl.BlockSpec((bn, D), lambda i: (i, 0)),
            _full_spec((8, D)), _full_spec((8, D)),
            _full_spec((1, D)), _full_spec((1, D)),
            _full_spec((D, 2 * D)), _full_spec((1, 2 * D)),
            _full_spec((2 * D, D)), _full_spec((1, D)),
        ],
        out_specs=[
            pl.BlockSpec((bn, D), lambda i: (i, 0)),
            _full_spec((8, D)), _full_spec((8, D)),
        ],
        out_shape=[
            jax.ShapeDtypeStruct((rows, D), fdt),
            jax.ShapeDtypeStruct((8, D), _F32),
            jax.ShapeDtypeStruct((8, D), _F32),
        ],
    )(y, sums, sqs, g, b, w1, b1, w2, b2)


def _post_c(f, sums, sqs, g, b, bn):
    rows = f.shape[0]
    grid = rows // bn
    return pl.pallas_call(
        functools.partial(_post_c_body, rows=float(rows)),
        grid=(grid,),
        in_specs=[
            pl.BlockSpec((bn, D), lambda i: (i, 0)),
            _full_spec((8, D)), _full_spec((8, D)),
            _full_spec((1, D)), _full_spec((1, D)),
        ],
        out_specs=pl.BlockSpec((bn, D), lambda i: (i, 0)),
        out_shape=jax.ShapeDtypeStruct((rows, D), _F32),
    )(f, sums, sqs, g, b)


def kernel(h, e, edge_index, WQ, WK, WV, WE, WOh, bOh, WOe, bOe,
           bn1h_g, bn1h_b, bn1e_g, bn1e_b, W1h, b1h, W2h, b2h,
           W1e, b1e, W2e, b2e, bn2h_g, bn2h_b, bn2e_g, bn2e_b):
    src = edge_index[0].astype(_I32)
    dst = edge_index[1].astype(_I32)
    scale = 1.0 / math.sqrt(DH)

    wq = WQ.T
    wk = WK.T * scale
    wv = WV.T
    wkqv = jnp.concatenate([wk, wq, wv], axis=1)

    kqv = _proj(h, wkqv, 1000)
    k2 = kqv[0:2].reshape(NC * N, HALF)
    q2 = kqv[2:4].reshape(NC * N, HALF)
    v2 = kqv[4:6].reshape(NC * N, HALF)
    ee2 = _proj(e, WE.T, 2000).reshape(NC * E * HALF)

    eout2, hatt2, w2 = _sc_edge(k2, q2, v2, ee2, src, dst)
    eout2 = eout2.reshape(NC * E, HALF)
    ssum2 = _sc_ssum(w2, src)

    y_h, s1h, q1h = _post_a_h(hatt2, ssum2, h, WOh.T,
                              bOh.reshape(1, D), 1000)
    f_h, s2h, q2h = _post_b(y_h, s1h, q1h, bn1h_g.reshape(1, D),
                            bn1h_b.reshape(1, D), W1h.T, b1h.reshape(1, 2 * D),
                            W2h.T, b2h.reshape(1, D), 1000)
    h_out = _post_c(f_h, s2h, q2h, bn2h_g.reshape(1, D),
                    bn2h_b.reshape(1, D), 1000)

    y_e, s1e, q1e = _post_a_e(eout2, e, WOe.T, bOe.reshape(1, D), 2000,
                              ydt=jnp.bfloat16)
    f_e, s2e, q2e = _post_b(y_e, s1e, q1e, bn1e_g.reshape(1, D),
                            bn1e_b.reshape(1, D), W1e.T, b1e.reshape(1, 2 * D),
                            W2e.T, b2e.reshape(1, D), 2000, fdt=jnp.bfloat16)
    e_out = _post_c(f_e, s2e, q2e, bn2e_g.reshape(1, D),
                    bn2e_b.reshape(1, D), 2000)

    return (h_out, e_out)
